# Initial kernel scaffold; baseline (speedup 1.0000x reference)
import jax
import jax.numpy as jnp
from jax.experimental import pallas as pl


def kernel(x, edge_attr, edge_index, emb_W, emb_b, Wm, bm, We, be, Weu, beu, node_W, node_b, ep_W, ep_b):
    raise NotImplementedError("write your pallas kernel here")


# speedup vs baseline: 1.5691x; 1.5691x over previous
import functools

import jax
import jax.numpy as jnp
from jax import lax
from jax.experimental import pallas as pl
from jax.experimental.pallas import tpu as pltpu
from jax.experimental.pallas import tpu_sc as plsc

N = 10000
E = 320000
IN = 128
EMB = 128
EDGE = 16
OUT = 128

NC, NS = 2, 16
NW = NC * NS
EPW = E // NW
C = 80
NPAD = 10240
ROWS_PT = NPAD // NS
ZCH = 80
LN = 16

_mesh = plsc.VectorSubcoreMesh(
    core_axis_name="c", subcore_axis_name="s", num_cores=NC, num_subcores=NS
)


@functools.partial(
    pl.kernel,
    out_type=jax.ShapeDtypeStruct((NC, NPAD, EMB), jnp.float32),
    mesh=_mesh,
    scratch_types=[
        pltpu.VMEM((C,), jnp.int32),
        pltpu.VMEM((C,), jnp.int32),
        pltpu.VMEM((C, EMB), jnp.float32),
        pltpu.VMEM((C, EMB), jnp.float32),
        pltpu.VMEM_SHARED((NPAD, EMB), jnp.float32),
        pltpu.SemaphoreType.DMA,
    ],
)
def _sc_msg_aggr(q_hbm, r_hbm, src_hbm, dst_hbm, out_hbm,
                 idx_s, idx_d, gbuf, rbuf, acc, sem):
    c = lax.axis_index("c")
    s = lax.axis_index("s")
    wid = c * NS + s
    ebase = wid * EPW
    row0 = s * ROWS_PT

    zero = jnp.zeros((LN,), jnp.float32)

    def zfill(i, _):
        e = i // (EMB // LN)
        j = (i % (EMB // LN)) * LN
        gbuf[e, pl.ds(j, LN)] = zero
        return 0

    lax.fori_loop(0, ZCH * (EMB // LN), zfill, 0)

    def zcopy(k, _):
        pltpu.sync_copy(gbuf.at[pl.ds(0, ZCH)],
                        acc.at[pl.ds(row0 + k * ZCH, ZCH)])
        return 0

    lax.fori_loop(0, ROWS_PT // ZCH, zcopy, 0)
    plsc.subcore_barrier()

    def body(k, _):
        e0 = ebase + k * C
        pltpu.sync_copy(src_hbm.at[pl.ds(e0, C)], idx_s)
        pltpu.sync_copy(dst_hbm.at[pl.ds(e0, C)], idx_d)
        pltpu.async_copy(q_hbm.at[idx_s], gbuf, sem).wait()
        pltpu.sync_copy(r_hbm.at[pl.ds(e0, C)], rbuf)

        def cw(i, _):
            e = i // (EMB // LN)
            j = (i % (EMB // LN)) * LN
            gbuf[e, pl.ds(j, LN)] = jnp.maximum(
                gbuf[e, pl.ds(j, LN)] + rbuf[e, pl.ds(j, LN)], 0.0)
            return 0

        lax.fori_loop(0, C * (EMB // LN), cw, 0)
        pltpu.sync_copy(gbuf, acc.at[idx_d], add=True)
        return 0

    lax.fori_loop(0, EPW // C, body, 0)
    plsc.subcore_barrier()

    def ocopy(k, _):
        r0 = row0 + k * ZCH
        pltpu.sync_copy(acc.at[pl.ds(r0, ZCH)], gbuf.at[pl.ds(0, ZCH)])
        pltpu.sync_copy(gbuf.at[pl.ds(0, ZCH)], out_hbm.at[c, pl.ds(r0, ZCH)])
        return 0

    lax.fori_loop(0, ROWS_PT // ZCH, ocopy, 0)


@functools.partial(
    pl.kernel,
    out_type=jax.ShapeDtypeStruct((E, 2 * EDGE), jnp.float32),
    mesh=_mesh,
    scratch_types=[
        pltpu.VMEM((C,), jnp.int32),
        pltpu.VMEM((C,), jnp.int32),
        pltpu.VMEM((C, EMB), jnp.float32),
        pltpu.VMEM((C, EMB), jnp.float32),
        pltpu.VMEM((C, EDGE), jnp.float32),
        pltpu.VMEM((C, 2 * EDGE), jnp.float32),
        pltpu.SemaphoreType.DMA,
        pltpu.SemaphoreType.DMA,
    ],
)
def _sc_edge_update(ts_hbm, td_hbm, c_hbm, src_hbm, dst_hbm, ea_hbm,
                    idx_s, idx_d, sbuf, dbuf, cbuf, eabuf,
                    sem_s, sem_d):
    cc = lax.axis_index("c")
    s = lax.axis_index("s")
    ebase = (cc * NS + s) * EPW

    def body(k, _):
        e0 = ebase + k * C
        pltpu.sync_copy(src_hbm.at[pl.ds(e0, C)], idx_s)
        pltpu.sync_copy(dst_hbm.at[pl.ds(e0, C)], idx_d)
        d1 = pltpu.async_copy(ts_hbm.at[idx_s], sbuf, sem_s)
        d2 = pltpu.async_copy(td_hbm.at[idx_d], dbuf, sem_d)
        pltpu.sync_copy(c_hbm.at[pl.ds(e0, C)], cbuf)
        d1.wait()
        d2.wait()

        def cw(e, _):
            eau = jnp.maximum(
                sbuf[e, pl.ds(0, LN)] + dbuf[e, pl.ds(0, LN)]
                + cbuf[e, pl.ds(0, LN)], 0.0)
            eabuf[e, pl.ds(0, LN)] = eau
            eabuf[e, pl.ds(LN, LN)] = (sbuf[e, pl.ds(LN, LN)]
                                       + dbuf[e, pl.ds(LN, LN)])
            return 0

        lax.fori_loop(0, C, cw, 0)
        pltpu.sync_copy(eabuf, ea_hbm.at[pl.ds(e0, C)])
        return 0

    lax.fori_loop(0, EPW // C, body, 0)


NB = 1000
EB = 2000


def _rows(bshape):
    return pl.BlockSpec(bshape, lambda i: (i,) + (0,) * (len(bshape) - 1))


def _whole(shape):
    return pl.BlockSpec(shape, lambda i: (0,) * len(shape))


def _tc_node0(x, emb_W, emb_b, Wq):
    def body(x_r, W_r, b_r, Wq_r, h_r, q_r):
        h = jnp.dot(x_r[...], W_r[...], preferred_element_type=jnp.float32) + b_r[...]
        h_r[...] = h
        q_r[...] = jnp.dot(h, Wq_r[...], preferred_element_type=jnp.float32)

    return pl.pallas_call(
        body,
        grid=(N // NB,),
        in_specs=[_rows((NB, IN)), _whole((IN, EMB)), _whole((1, EMB)),
                  _whole((EMB, EMB))],
        out_specs=[_rows((NB, EMB)), _rows((NB, EMB))],
        out_shape=[jax.ShapeDtypeStruct((N, EMB), jnp.float32)] * 2,
    )(x, emb_W, emb_b.reshape(1, -1), Wq)


def _tc_edge_pre(ea, Wme, bm, Wce, bc):
    W = ea.shape[1]

    def body(ea_r, Wm_r, bm_r, Wc_r, bc_r, r_r, c_r):
        a = ea_r[:, :EDGE]
        r_r[...] = jnp.dot(a, Wm_r[...],
                           preferred_element_type=jnp.float32) + bm_r[...]
        c_r[...] = jnp.dot(a, Wc_r[...],
                           preferred_element_type=jnp.float32) + bc_r[...]

    return pl.pallas_call(
        body,
        grid=(E // EB,),
        in_specs=[_rows((EB, W)), _whole((EDGE, EMB)), _whole((1, EMB)),
                  _whole((EDGE, EDGE)), _whole((1, EDGE))],
        out_specs=[_rows((EB, EMB)), _rows((EB, EDGE))],
        out_shape=[jax.ShapeDtypeStruct((E, EMB), jnp.float32),
                   jax.ShapeDtypeStruct((E, EDGE), jnp.float32)],
    )(ea, Wme, bm.reshape(1, -1), Wce, bc.reshape(1, -1))


def _tc_node_update(aggr, h, We_a, We_h, be, Wq, Wts, Wtd):
    def body(a0_r, a1_r, h_r, Wa_r, Wh_r, b_r, Wq_r, Ws_r, Wd_r,
             hn_r, q_r, Ts_r, Td_r):
        a = a0_r[...] + a1_r[...]
        hn = jnp.maximum(
            jnp.dot(a, Wa_r[...], preferred_element_type=jnp.float32)
            + jnp.dot(h_r[...], Wh_r[...], preferred_element_type=jnp.float32)
            + b_r[...], 0.0)
        hn_r[...] = hn
        q_r[...] = jnp.dot(hn, Wq_r[...], preferred_element_type=jnp.float32)
        Ts_r[...] = jnp.dot(hn, Ws_r[...], preferred_element_type=jnp.float32)
        Td_r[...] = jnp.dot(hn, Wd_r[...], preferred_element_type=jnp.float32)

    return pl.pallas_call(
        body,
        grid=(N // NB,),
        in_specs=[_rows((NB, EMB)), _rows((NB, EMB)), _rows((NB, EMB)),
                  _whole((EMB, EMB)), _whole((EMB, EMB)), _whole((1, EMB)),
                  _whole((EMB, EMB)), _whole((EMB, EMB)), _whole((EMB, EMB))],
        out_specs=[_rows((NB, EMB)), _rows((NB, EMB)),
                   _rows((NB, EMB)), _rows((NB, EMB))],
        out_shape=[jax.ShapeDtypeStruct((N, EMB), jnp.float32)] * 4,
    )(aggr[0], aggr[1], h, We_a, We_h, be.reshape(1, -1), Wq, Wts, Wtd)


def _tc_node_final(aggr, h, We_a, We_h, be, node_W, node_b, Wts, Wtd):
    def body(a0_r, a1_r, h_r, Wa_r, Wh_r, b_r, Wn_r, bn_r, Ws_r, Wd_r,
             np_r, Ts_r, Td_r):
        a = a0_r[...] + a1_r[...]
        h2 = jnp.maximum(
            jnp.dot(a, Wa_r[...], preferred_element_type=jnp.float32)
            + jnp.dot(h_r[...], Wh_r[...], preferred_element_type=jnp.float32)
            + b_r[...], 0.0)
        np_r[...] = jnp.dot(h2, Wn_r[...],
                            preferred_element_type=jnp.float32) + bn_r[...]
        Ts_r[...] = jnp.dot(h2, Ws_r[...], preferred_element_type=jnp.float32)
        Td_r[...] = jnp.dot(h2, Wd_r[...], preferred_element_type=jnp.float32)

    return pl.pallas_call(
        body,
        grid=(N // NB,),
        in_specs=[_rows((NB, EMB)), _rows((NB, EMB)), _rows((NB, EMB)),
                  _whole((EMB, EMB)), _whole((EMB, EMB)), _whole((1, EMB)),
                  _whole((EMB, OUT)), _whole((1, OUT)),
                  _whole((EMB, EMB)), _whole((EMB, EMB))],
        out_specs=[_rows((NB, OUT)), _rows((NB, EMB)), _rows((NB, EMB))],
        out_shape=[jax.ShapeDtypeStruct((N, OUT), jnp.float32),
                   jax.ShapeDtypeStruct((N, EMB), jnp.float32),
                   jax.ShapeDtypeStruct((N, EMB), jnp.float32)],
    )(aggr[0], aggr[1], h, We_a, We_h, be.reshape(1, -1),
      node_W, node_b.reshape(1, -1), Wts, Wtd)


def _tc_edge_final(ea2e, ep_We, ep_b):
    def body(ea_r, Wp_r, bp_r, ep_r):
        ep_r[...] = (jnp.dot(ea_r[:, :EDGE], Wp_r[...],
                             preferred_element_type=jnp.float32)
                     + ea_r[:, EDGE:EDGE + 1] + bp_r[...])

    return pl.pallas_call(
        body,
        grid=(E // EB,),
        in_specs=[_rows((EB, 2 * EDGE)), _whole((EDGE, 1)), _whole((1, 1))],
        out_specs=_rows((EB, 1)),
        out_shape=jax.ShapeDtypeStruct((E, 1), jnp.float32),
    )(ea2e, ep_We, ep_b.reshape(1, 1))


def kernel(x, edge_attr, edge_index, emb_W, emb_b, Wm, bm, We, be,
           Weu, beu, node_W, node_b, ep_W, ep_b):
    src = edge_index[0]
    dst = edge_index[1]

    Wm_h = [Wm[i, :EMB] for i in range(2)]
    Wm_e = [Wm[i, EMB:] for i in range(2)]
    We_a = [We[i, :EMB] for i in range(2)]
    We_h = [We[i, EMB:] for i in range(2)]
    Weu_s = [Weu[i, :EMB] for i in range(2)]
    Weu_d = [Weu[i, EMB:2 * EMB] for i in range(2)]
    Weu_e = [Weu[i, 2 * EMB:] for i in range(2)]
    zcol = jnp.zeros((EMB, 1), jnp.float32)
    zpad = jnp.zeros((EMB, EMB - EDGE - 1), jnp.float32)
    Wts1 = jnp.concatenate([Weu_s[0], zcol, zpad], axis=1)
    Wtd1 = jnp.concatenate([Weu_d[0], zcol, zpad], axis=1)
    Wts2 = jnp.concatenate([Weu_s[1], ep_W[:EMB], zpad], axis=1)
    Wtd2 = jnp.concatenate([Weu_d[1], ep_W[EMB:2 * EMB], zpad], axis=1)
    ep_We = ep_W[2 * EMB:]

    h0, q0 = _tc_node0(x, emb_W, emb_b, Wm_h[0])
    r0, c0 = _tc_edge_pre(edge_attr, Wm_e[0], bm[0], Weu_e[0], beu[0])
    aggr0 = _sc_msg_aggr(q0, r0, src, dst)
    h1, q1, T1s, T1d = _tc_node_update(aggr0, h0, We_a[0], We_h[0], be[0],
                                       Wm_h[1], Wts1, Wtd1)
    ea1e = _sc_edge_update(T1s, T1d, c0, src, dst)

    r1, c1 = _tc_edge_pre(ea1e, Wm_e[1], bm[1], Weu_e[1], beu[1])
    aggr1 = _sc_msg_aggr(q1, r1, src, dst)
    node_pred, T2s, T2d = _tc_node_final(aggr1, h1, We_a[1], We_h[1], be[1],
                                         node_W, node_b, Wts2, Wtd2)
    ea2e = _sc_edge_update(T2s, T2d, c1, src, dst)
    edge_pred = _tc_edge_final(ea2e, ep_We, ep_b)

    return (node_pred, edge_pred)


# speedup vs baseline: 2.3167x; 1.4765x over previous
import functools

import jax
import jax.numpy as jnp
from jax import lax
from jax.experimental import pallas as pl
from jax.experimental.pallas import tpu as pltpu
from jax.experimental.pallas import tpu_sc as plsc

N = 10000
E = 320000
IN = 128
EMB = 128
EDGE = 16
OUT = 128

NC, NS = 2, 16
NW = NC * NS
EPW = E // NW
C = 80
NCH = EPW // C
NPAD = 10240
ROWS_PT = NPAD // NS
LN = 16

_mesh = plsc.VectorSubcoreMesh(
    core_axis_name="c", subcore_axis_name="s", num_cores=NC, num_subcores=NS
)


@functools.partial(
    pl.kernel,
    out_type=jax.ShapeDtypeStruct((NC, NPAD, EMB), jnp.float32),
    mesh=_mesh,
    scratch_types=[
        pltpu.VMEM((C,), jnp.int32),
        pltpu.VMEM((C,), jnp.int32),
        pltpu.VMEM((C,), jnp.int32),
        pltpu.VMEM((C,), jnp.int32),
        pltpu.VMEM((C, EMB), jnp.float32),
        pltpu.VMEM((C, EMB), jnp.float32),
        pltpu.VMEM((C, EMB), jnp.float32),
        pltpu.VMEM((C, EMB), jnp.float32),
        pltpu.VMEM_SHARED((NPAD, EMB), jnp.float32),
        pltpu.SemaphoreType.DMA,
        pltpu.SemaphoreType.DMA,
        pltpu.SemaphoreType.DMA,
        pltpu.SemaphoreType.DMA,
        pltpu.SemaphoreType.DMA,
        pltpu.SemaphoreType.DMA,
    ],
)
def _sc_msg_aggr(q_hbm, r_hbm, src_hbm, dst_hbm, out_hbm,
                 ids0, ids1, idd0, idd1, gbuf0, gbuf1, rbuf0, rbuf1, acc,
                 si0, si1, sg0, sg1, sr0, sr1):
    c = lax.axis_index("c")
    s = lax.axis_index("s")
    wid = c * NS + s
    ebase = wid * EPW
    row0 = s * ROWS_PT
    ids, idd = (ids0, ids1), (idd0, idd1)
    gbufs, rbufs = (gbuf0, gbuf1), (rbuf0, rbuf1)
    sis, sgs, srs = (si0, si1), (sg0, sg1), (sr0, sr1)

    def _issue_idx(k, b):
        pltpu.async_copy(src_hbm.at[pl.ds(ebase + k * C, C)], ids[b], sis[b])
        pltpu.async_copy(dst_hbm.at[pl.ds(ebase + k * C, C)], idd[b], sis[b])

    def _wait_idx(k, b):
        pltpu.make_async_copy(src_hbm.at[pl.ds(ebase + k * C, C)], ids[b],
                              sis[b]).wait()
        pltpu.make_async_copy(dst_hbm.at[pl.ds(ebase + k * C, C)], idd[b],
                              sis[b]).wait()

    zero = jnp.zeros((LN,), jnp.float32)

    def zfill(i, _):
        e = i // (EMB // LN)
        j = (i % (EMB // LN)) * LN
        gbuf0[e, pl.ds(j, LN)] = zero
        return 0

    lax.fori_loop(0, C * (EMB // LN), zfill, 0, unroll=4)

    def zissue(k, _):
        pltpu.async_copy(gbuf0, acc.at[pl.ds(row0 + k * C, C)], sg0)
        return 0

    lax.fori_loop(0, ROWS_PT // C, zissue, 0)

    def zdrain(k, _):
        pltpu.make_async_copy(gbuf0, acc.at[pl.ds(row0 + k * C, C)],
                              sg0).wait()
        return 0

    lax.fori_loop(0, ROWS_PT // C, zdrain, 0)
    plsc.subcore_barrier()

    def _issue_data(k, b):
        pltpu.async_copy(q_hbm.at[ids[b]], gbufs[b], sgs[b])
        pltpu.async_copy(r_hbm.at[pl.ds(ebase + k * C, C)], rbufs[b], srs[b])

    def _step(k, b, tail=False):
        if not tail:
            _wait_idx(k + 1, 1 - b)
            _issue_data(k + 1, 1 - b)

        pltpu.make_async_copy(q_hbm.at[ids[b]], gbufs[b], sgs[b]).wait()
        pltpu.make_async_copy(r_hbm.at[pl.ds(ebase + k * C, C)], rbufs[b],
                              srs[b]).wait()
        g, r = gbufs[b], rbufs[b]

        def cw(e, _):
            for j in range(EMB // LN):
                g[e, pl.ds(j * LN, LN)] = jnp.maximum(
                    g[e, pl.ds(j * LN, LN)] + r[e, pl.ds(j * LN, LN)], 0.0)
            return 0

        lax.fori_loop(0, C, cw, 0, unroll=2)
        pltpu.sync_copy(g, acc.at[idd[b]], add=True)

        if not tail:
            @pl.when(k + 2 < NCH)
            def _():
                _issue_idx(k + 2, b)

    _issue_idx(0, 0)
    _issue_idx(1, 1)
    _wait_idx(0, 0)
    _issue_data(0, 0)

    def body(k2, _):
        _step(2 * k2, 0)
        _step(2 * k2 + 1, 1)
        return 0

    lax.fori_loop(0, NCH // 2, body, 0)
    _step(NCH - 1, (NCH - 1) % 2, tail=True)
    plsc.subcore_barrier()

    def ocopy(k, _):
        r0 = row0 + k * C
        pltpu.sync_copy(acc.at[pl.ds(r0, C)], gbuf0)
        pltpu.sync_copy(gbuf0, out_hbm.at[c, pl.ds(r0, C)])
        return 0

    lax.fori_loop(0, ROWS_PT // C, ocopy, 0)


@functools.partial(
    pl.kernel,
    out_type=jax.ShapeDtypeStruct((E, 2 * EDGE), jnp.float32),
    mesh=_mesh,
    scratch_types=[
        pltpu.VMEM((EPW,), jnp.int32),
        pltpu.VMEM((EPW,), jnp.int32),
        pltpu.VMEM((C, EMB), jnp.float32),
        pltpu.VMEM((C, EMB), jnp.float32),
        pltpu.VMEM((C, EMB), jnp.float32),
        pltpu.VMEM((C, EMB), jnp.float32),
        pltpu.VMEM((C, EDGE), jnp.float32),
        pltpu.VMEM((C, EDGE), jnp.float32),
        pltpu.VMEM((C, 2 * EDGE), jnp.float32),
        pltpu.VMEM((C, 2 * EDGE), jnp.float32),
        pltpu.SemaphoreType.DMA,
        pltpu.SemaphoreType.DMA,
        pltpu.SemaphoreType.DMA,
        pltpu.SemaphoreType.DMA,
        pltpu.SemaphoreType.DMA,
        pltpu.SemaphoreType.DMA,
    ],
)
def _sc_edge_update(ts_hbm, td_hbm, c_hbm, src_hbm, dst_hbm, ea_hbm,
                    idx_s, idx_d, sbuf0, sbuf1, dbuf0, dbuf1,
                    cbuf0, cbuf1, eabuf0, eabuf1,
                    ss0, ss1, sd0, sd1, sc0, sc1):
    cc = lax.axis_index("c")
    s = lax.axis_index("s")
    wid = cc * NS + s
    ebase = wid * EPW
    sbufs, dbufs = (sbuf0, sbuf1), (dbuf0, dbuf1)
    cbufs, eabufs = (cbuf0, cbuf1), (eabuf0, eabuf1)
    sss, sds, scs = (ss0, ss1), (sd0, sd1), (sc0, sc1)

    pltpu.sync_copy(src_hbm.at[pl.ds(ebase, EPW)], idx_s)
    pltpu.sync_copy(dst_hbm.at[pl.ds(ebase, EPW)], idx_d)

    def _issue(k, b):
        pltpu.async_copy(ts_hbm.at[idx_s.at[pl.ds(k * C, C)]], sbufs[b],
                         sss[b])
        pltpu.async_copy(td_hbm.at[idx_d.at[pl.ds(k * C, C)]], dbufs[b],
                         sds[b])
        pltpu.async_copy(c_hbm.at[pl.ds(ebase + k * C, C)], cbufs[b], scs[b])

    def _step(k, b, tail=False):
        if not tail:
            _issue(k + 1, 1 - b)

        pltpu.make_async_copy(ts_hbm.at[idx_s.at[pl.ds(k * C, C)]], sbufs[b],
                              sss[b]).wait()
        pltpu.make_async_copy(td_hbm.at[idx_d.at[pl.ds(k * C, C)]], dbufs[b],
                              sds[b]).wait()
        pltpu.make_async_copy(c_hbm.at[pl.ds(ebase + k * C, C)], cbufs[b],
                              scs[b]).wait()
        sb, db, cb, eb = sbufs[b], dbufs[b], cbufs[b], eabufs[b]

        def cw(e, _):
            eb[e, pl.ds(0, LN)] = jnp.maximum(
                sb[e, pl.ds(0, LN)] + db[e, pl.ds(0, LN)]
                + cb[e, pl.ds(0, LN)], 0.0)
            eb[e, pl.ds(LN, LN)] = (sb[e, pl.ds(LN, LN)]
                                    + db[e, pl.ds(LN, LN)])
            return 0

        lax.fori_loop(0, C, cw, 0, unroll=4)
        pltpu.sync_copy(eb, ea_hbm.at[pl.ds(ebase + k * C, C)])

    _issue(0, 0)

    def body(k2, _):
        _step(2 * k2, 0)
        _step(2 * k2 + 1, 1)
        return 0

    lax.fori_loop(0, NCH // 2, body, 0)
    _step(NCH - 1, (NCH - 1) % 2, tail=True)


NB = 1000
EB = 2000


def _rows(bshape):
    return pl.BlockSpec(bshape, lambda i: (i,) + (0,) * (len(bshape) - 1))


def _whole(shape):
    return pl.BlockSpec(shape, lambda i: (0,) * len(shape))


def _tc_node0(x, emb_W, emb_b, Wq):
    def body(x_r, W_r, b_r, Wq_r, h_r, q_r):
        h = jnp.dot(x_r[...], W_r[...], preferred_element_type=jnp.float32) + b_r[...]
        h_r[...] = h
        q_r[...] = jnp.dot(h, Wq_r[...], preferred_element_type=jnp.float32)

    return pl.pallas_call(
        body,
        grid=(N // NB,),
        in_specs=[_rows((NB, IN)), _whole((IN, EMB)), _whole((1, EMB)),
                  _whole((EMB, EMB))],
        out_specs=[_rows((NB, EMB)), _rows((NB, EMB))],
        out_shape=[jax.ShapeDtypeStruct((N, EMB), jnp.float32)] * 2,
    )(x, emb_W, emb_b.reshape(1, -1), Wq)


def _tc_edge_pre(ea, Wme, bm, Wce, bc):
    W = ea.shape[1]

    def body(ea_r, Wm_r, bm_r, Wc_r, bc_r, r_r, c_r):
        a = ea_r[:, :EDGE]
        r_r[...] = jnp.dot(a, Wm_r[...],
                           preferred_element_type=jnp.float32) + bm_r[...]
        c_r[...] = jnp.dot(a, Wc_r[...],
                           preferred_element_type=jnp.float32) + bc_r[...]

    return pl.pallas_call(
        body,
        grid=(E // EB,),
        in_specs=[_rows((EB, W)), _whole((EDGE, EMB)), _whole((1, EMB)),
                  _whole((EDGE, EDGE)), _whole((1, EDGE))],
        out_specs=[_rows((EB, EMB)), _rows((EB, EDGE))],
        out_shape=[jax.ShapeDtypeStruct((E, EMB), jnp.float32),
                   jax.ShapeDtypeStruct((E, EDGE), jnp.float32)],
    )(ea, Wme, bm.reshape(1, -1), Wce, bc.reshape(1, -1))


def _tc_node_update(aggr, h, We_a, We_h, be, Wq, Wts, Wtd):
    def body(a0_r, a1_r, h_r, Wa_r, Wh_r, b_r, Wq_r, Ws_r, Wd_r,
             hn_r, q_r, Ts_r, Td_r):
        a = a0_r[...] + a1_r[...]
        hn = jnp.maximum(
            jnp.dot(a, Wa_r[...], preferred_element_type=jnp.float32)
            + jnp.dot(h_r[...], Wh_r[...], preferred_element_type=jnp.float32)
            + b_r[...], 0.0)
        hn_r[...] = hn
        q_r[...] = jnp.dot(hn, Wq_r[...], preferred_element_type=jnp.float32)
        Ts_r[...] = jnp.dot(hn, Ws_r[...], preferred_element_type=jnp.float32)
        Td_r[...] = jnp.dot(hn, Wd_r[...], preferred_element_type=jnp.float32)

    return pl.pallas_call(
        body,
        grid=(N // NB,),
        in_specs=[_rows((NB, EMB)), _rows((NB, EMB)), _rows((NB, EMB)),
                  _whole((EMB, EMB)), _whole((EMB, EMB)), _whole((1, EMB)),
                  _whole((EMB, EMB)), _whole((EMB, EMB)), _whole((EMB, EMB))],
        out_specs=[_rows((NB, EMB)), _rows((NB, EMB)),
                   _rows((NB, EMB)), _rows((NB, EMB))],
        out_shape=[jax.ShapeDtypeStruct((N, EMB), jnp.float32)] * 4,
    )(aggr[0], aggr[1], h, We_a, We_h, be.reshape(1, -1), Wq, Wts, Wtd)


def _tc_node_final(aggr, h, We_a, We_h, be, node_W, node_b, Wts, Wtd):
    def body(a0_r, a1_r, h_r, Wa_r, Wh_r, b_r, Wn_r, bn_r, Ws_r, Wd_r,
             np_r, Ts_r, Td_r):
        a = a0_r[...] + a1_r[...]
        h2 = jnp.maximum(
            jnp.dot(a, Wa_r[...], preferred_element_type=jnp.float32)
            + jnp.dot(h_r[...], Wh_r[...], preferred_element_type=jnp.float32)
            + b_r[...], 0.0)
        np_r[...] = jnp.dot(h2, Wn_r[...],
                            preferred_element_type=jnp.float32) + bn_r[...]
        Ts_r[...] = jnp.dot(h2, Ws_r[...], preferred_element_type=jnp.float32)
        Td_r[...] = jnp.dot(h2, Wd_r[...], preferred_element_type=jnp.float32)

    return pl.pallas_call(
        body,
        grid=(N // NB,),
        in_specs=[_rows((NB, EMB)), _rows((NB, EMB)), _rows((NB, EMB)),
                  _whole((EMB, EMB)), _whole((EMB, EMB)), _whole((1, EMB)),
                  _whole((EMB, OUT)), _whole((1, OUT)),
                  _whole((EMB, EMB)), _whole((EMB, EMB))],
        out_specs=[_rows((NB, OUT)), _rows((NB, EMB)), _rows((NB, EMB))],
        out_shape=[jax.ShapeDtypeStruct((N, OUT), jnp.float32),
                   jax.ShapeDtypeStruct((N, EMB), jnp.float32),
                   jax.ShapeDtypeStruct((N, EMB), jnp.float32)],
    )(aggr[0], aggr[1], h, We_a, We_h, be.reshape(1, -1),
      node_W, node_b.reshape(1, -1), Wts, Wtd)


def _tc_edge_final(ea2e, ep_We, ep_b):
    def body(ea_r, Wp_r, bp_r, ep_r):
        ep_r[...] = (jnp.dot(ea_r[:, :EDGE], Wp_r[...],
                             preferred_element_type=jnp.float32)
                     + ea_r[:, EDGE:EDGE + 1] + bp_r[...])

    return pl.pallas_call(
        body,
        grid=(E // EB,),
        in_specs=[_rows((EB, 2 * EDGE)), _whole((EDGE, 1)), _whole((1, 1))],
        out_specs=_rows((EB, 1)),
        out_shape=jax.ShapeDtypeStruct((E, 1), jnp.float32),
    )(ea2e, ep_We, ep_b.reshape(1, 1))


def kernel(x, edge_attr, edge_index, emb_W, emb_b, Wm, bm, We, be,
           Weu, beu, node_W, node_b, ep_W, ep_b):
    src = edge_index[0]
    dst = edge_index[1]

    Wm_h = [Wm[i, :EMB] for i in range(2)]
    Wm_e = [Wm[i, EMB:] for i in range(2)]
    We_a = [We[i, :EMB] for i in range(2)]
    We_h = [We[i, EMB:] for i in range(2)]
    Weu_s = [Weu[i, :EMB] for i in range(2)]
    Weu_d = [Weu[i, EMB:2 * EMB] for i in range(2)]
    Weu_e = [Weu[i, 2 * EMB:] for i in range(2)]
    zcol = jnp.zeros((EMB, 1), jnp.float32)
    zpad = jnp.zeros((EMB, EMB - EDGE - 1), jnp.float32)
    Wts1 = jnp.concatenate([Weu_s[0], zcol, zpad], axis=1)
    Wtd1 = jnp.concatenate([Weu_d[0], zcol, zpad], axis=1)
    Wts2 = jnp.concatenate([Weu_s[1], ep_W[:EMB], zpad], axis=1)
    Wtd2 = jnp.concatenate([Weu_d[1], ep_W[EMB:2 * EMB], zpad], axis=1)
    ep_We = ep_W[2 * EMB:]

    h0, q0 = _tc_node0(x, emb_W, emb_b, Wm_h[0])
    r0, c0 = _tc_edge_pre(edge_attr, Wm_e[0], bm[0], Weu_e[0], beu[0])
    aggr0 = _sc_msg_aggr(q0, r0, src, dst)
    h1, q1, T1s, T1d = _tc_node_update(aggr0, h0, We_a[0], We_h[0], be[0],
                                       Wm_h[1], Wts1, Wtd1)
    ea1e = _sc_edge_update(T1s, T1d, c0, src, dst)

    r1, c1 = _tc_edge_pre(ea1e, Wm_e[1], bm[1], Weu_e[1], beu[1])
    aggr1 = _sc_msg_aggr(q1, r1, src, dst)
    node_pred, T2s, T2d = _tc_node_final(aggr1, h1, We_a[1], We_h[1], be[1],
                                         node_W, node_b, Wts2, Wtd2)
    ea2e = _sc_edge_update(T2s, T2d, c1, src, dst)
    edge_pred = _tc_edge_final(ea2e, ep_We, ep_b)

    return (node_pred, edge_pred)


# speedup vs baseline: 3.1603x; 1.3642x over previous
import functools

import jax
import jax.numpy as jnp
from jax import lax
from jax.experimental import pallas as pl
from jax.experimental.pallas import tpu as pltpu
from jax.experimental.pallas import tpu_sc as plsc

N = 10000
E = 320000
IN = 128
EMB = 128
EDGE = 16
OUT = 128

NC, NS = 2, 16
NW = NC * NS
EPW = E // NW
C = 80
NCH = EPW // C
NPAD = 10240
ROWS_PT = NPAD // NS
LN = 16

_mesh = plsc.VectorSubcoreMesh(
    core_axis_name="c", subcore_axis_name="s", num_cores=NC, num_subcores=NS
)


@functools.partial(
    pl.kernel,
    out_type=jax.ShapeDtypeStruct((NC, NPAD, EMB), jnp.float32),
    mesh=_mesh,
    scratch_types=[
        pltpu.VMEM((C,), jnp.int32),
        pltpu.VMEM((C,), jnp.int32),
        pltpu.VMEM((C,), jnp.int32),
        pltpu.VMEM((C,), jnp.int32),
        pltpu.VMEM((C, EMB), jnp.float32),
        pltpu.VMEM((C, EMB), jnp.float32),
        pltpu.VMEM((C, EMB), jnp.float32),
        pltpu.VMEM((C, EMB), jnp.float32),
        pltpu.VMEM_SHARED((NPAD, EMB), jnp.float32),
        pltpu.SemaphoreType.DMA,
        pltpu.SemaphoreType.DMA,
        pltpu.SemaphoreType.DMA,
        pltpu.SemaphoreType.DMA,
        pltpu.SemaphoreType.DMA,
        pltpu.SemaphoreType.DMA,
    ],
)
def _sc_msg_aggr(q_hbm, r_hbm, src_hbm, dst_hbm, out_hbm,
                 ids0, ids1, idd0, idd1, gbuf0, gbuf1, rbuf0, rbuf1, acc,
                 si0, si1, sg0, sg1, sr0, sr1):
    c = lax.axis_index("c")
    s = lax.axis_index("s")
    wid = c * NS + s
    ebase = wid * EPW
    row0 = s * ROWS_PT
    ids, idd = (ids0, ids1), (idd0, idd1)
    gbufs, rbufs = (gbuf0, gbuf1), (rbuf0, rbuf1)
    sis, sgs, srs = (si0, si1), (sg0, sg1), (sr0, sr1)

    def _issue_idx(k, b):
        pltpu.async_copy(src_hbm.at[pl.ds(ebase + k * C, C)], ids[b], sis[b])
        pltpu.async_copy(dst_hbm.at[pl.ds(ebase + k * C, C)], idd[b], sis[b])

    def _wait_idx(k, b):
        pltpu.make_async_copy(src_hbm.at[pl.ds(ebase + k * C, C)], ids[b],
                              sis[b]).wait()
        pltpu.make_async_copy(dst_hbm.at[pl.ds(ebase + k * C, C)], idd[b],
                              sis[b]).wait()

    zero = jnp.zeros((LN,), jnp.float32)

    def zfill(i, _):
        e = i // (EMB // LN)
        j = (i % (EMB // LN)) * LN
        gbuf0[e, pl.ds(j, LN)] = zero
        return 0

    lax.fori_loop(0, C * (EMB // LN), zfill, 0, unroll=4)

    def zissue(k, _):
        pltpu.async_copy(gbuf0, acc.at[pl.ds(row0 + k * C, C)], sg0)
        return 0

    lax.fori_loop(0, ROWS_PT // C, zissue, 0)

    def zdrain(k, _):
        pltpu.make_async_copy(gbuf0, acc.at[pl.ds(row0 + k * C, C)],
                              sg0).wait()
        return 0

    lax.fori_loop(0, ROWS_PT // C, zdrain, 0)
    plsc.subcore_barrier()

    def _issue_data(k, b):
        pltpu.async_copy(q_hbm.at[ids[b]], gbufs[b], sgs[b])
        pltpu.async_copy(r_hbm.at[pl.ds(ebase + k * C, C)], rbufs[b], srs[b])

    def _step(k, b, tail=False):
        if not tail:
            _wait_idx(k + 1, 1 - b)
            _issue_data(k + 1, 1 - b)

        pltpu.make_async_copy(q_hbm.at[ids[b]], gbufs[b], sgs[b]).wait()
        pltpu.make_async_copy(r_hbm.at[pl.ds(ebase + k * C, C)], rbufs[b],
                              srs[b]).wait()
        g, r = gbufs[b], rbufs[b]

        @plsc.parallel_loop(0, C, 1, unroll=4)
        def cw(e):
            for j in range(EMB // LN):
                g[e, pl.ds(j * LN, LN)] = jnp.maximum(
                    g[e, pl.ds(j * LN, LN)] + r[e, pl.ds(j * LN, LN)], 0.0)

        pltpu.sync_copy(g, acc.at[idd[b]], add=True)

        if not tail:
            @pl.when(k + 2 < NCH)
            def _():
                _issue_idx(k + 2, b)

    _issue_idx(0, 0)
    _issue_idx(1, 1)
    _wait_idx(0, 0)
    _issue_data(0, 0)

    def body(k2, _):
        _step(2 * k2, 0)
        _step(2 * k2 + 1, 1)
        return 0

    lax.fori_loop(0, NCH // 2, body, 0)
    _step(NCH - 1, (NCH - 1) % 2, tail=True)
    plsc.subcore_barrier()

    def ocopy(k, _):
        r0 = row0 + k * C
        pltpu.sync_copy(acc.at[pl.ds(r0, C)], gbuf0)
        pltpu.sync_copy(gbuf0, out_hbm.at[c, pl.ds(r0, C)])
        return 0

    lax.fori_loop(0, ROWS_PT // C, ocopy, 0)


@functools.partial(
    pl.kernel,
    out_type=jax.ShapeDtypeStruct((E, 2 * EDGE), jnp.float32),
    mesh=_mesh,
    scratch_types=[
        pltpu.VMEM((EPW,), jnp.int32),
        pltpu.VMEM((EPW,), jnp.int32),
        pltpu.VMEM((C, EMB), jnp.float32),
        pltpu.VMEM((C, EMB), jnp.float32),
        pltpu.VMEM((C, EMB), jnp.float32),
        pltpu.VMEM((C, EMB), jnp.float32),
        pltpu.VMEM((C, EDGE), jnp.float32),
        pltpu.VMEM((C, EDGE), jnp.float32),
        pltpu.VMEM((C, 2 * EDGE), jnp.float32),
        pltpu.VMEM((C, 2 * EDGE), jnp.float32),
        pltpu.SemaphoreType.DMA,
        pltpu.SemaphoreType.DMA,
        pltpu.SemaphoreType.DMA,
        pltpu.SemaphoreType.DMA,
        pltpu.SemaphoreType.DMA,
        pltpu.SemaphoreType.DMA,
    ],
)
def _sc_edge_update(ts_hbm, td_hbm, c_hbm, src_hbm, dst_hbm, ea_hbm,
                    idx_s, idx_d, sbuf0, sbuf1, dbuf0, dbuf1,
                    cbuf0, cbuf1, eabuf0, eabuf1,
                    ss0, ss1, sd0, sd1, sc0, sc1):
    cc = lax.axis_index("c")
    s = lax.axis_index("s")
    wid = cc * NS + s
    ebase = wid * EPW
    sbufs, dbufs = (sbuf0, sbuf1), (dbuf0, dbuf1)
    cbufs, eabufs = (cbuf0, cbuf1), (eabuf0, eabuf1)
    sss, sds, scs = (ss0, ss1), (sd0, sd1), (sc0, sc1)

    pltpu.sync_copy(src_hbm.at[pl.ds(ebase, EPW)], idx_s)
    pltpu.sync_copy(dst_hbm.at[pl.ds(ebase, EPW)], idx_d)

    def _issue(k, b):
        pltpu.async_copy(ts_hbm.at[idx_s.at[pl.ds(k * C, C)]], sbufs[b],
                         sss[b])
        pltpu.async_copy(td_hbm.at[idx_d.at[pl.ds(k * C, C)]], dbufs[b],
                         sds[b])
        pltpu.async_copy(c_hbm.at[pl.ds(ebase + k * C, C)], cbufs[b], scs[b])

    def _step(k, b, tail=False):
        if not tail:
            _issue(k + 1, 1 - b)

        pltpu.make_async_copy(ts_hbm.at[idx_s.at[pl.ds(k * C, C)]], sbufs[b],
                              sss[b]).wait()
        pltpu.make_async_copy(td_hbm.at[idx_d.at[pl.ds(k * C, C)]], dbufs[b],
                              sds[b]).wait()
        pltpu.make_async_copy(c_hbm.at[pl.ds(ebase + k * C, C)], cbufs[b],
                              scs[b]).wait()
        sb, db, cb, eb = sbufs[b], dbufs[b], cbufs[b], eabufs[b]

        @plsc.parallel_loop(0, C, 1, unroll=4)
        def cw(e):
            eb[e, pl.ds(0, LN)] = jnp.maximum(
                sb[e, pl.ds(0, LN)] + db[e, pl.ds(0, LN)]
                + cb[e, pl.ds(0, LN)], 0.0)
            eb[e, pl.ds(LN, LN)] = (sb[e, pl.ds(LN, LN)]
                                    + db[e, pl.ds(LN, LN)])
        pltpu.sync_copy(eb, ea_hbm.at[pl.ds(ebase + k * C, C)])

    _issue(0, 0)

    def body(k2, _):
        _step(2 * k2, 0)
        _step(2 * k2 + 1, 1)
        return 0

    lax.fori_loop(0, NCH // 2, body, 0)
    _step(NCH - 1, (NCH - 1) % 2, tail=True)


NB = 1000
EB = 2000


def _rows(bshape):
    return pl.BlockSpec(bshape, lambda i: (i,) + (0,) * (len(bshape) - 1))


def _whole(shape):
    return pl.BlockSpec(shape, lambda i: (0,) * len(shape))


def _tc_node0(x, emb_W, emb_b, Wq):
    def body(x_r, W_r, b_r, Wq_r, h_r, q_r):
        h = jnp.dot(x_r[...], W_r[...], preferred_element_type=jnp.float32) + b_r[...]
        h_r[...] = h
        q_r[...] = jnp.dot(h, Wq_r[...], preferred_element_type=jnp.float32)

    return pl.pallas_call(
        body,
        grid=(N // NB,),
        in_specs=[_rows((NB, IN)), _whole((IN, EMB)), _whole((1, EMB)),
                  _whole((EMB, EMB))],
        out_specs=[_rows((NB, EMB)), _rows((NB, EMB))],
        out_shape=[jax.ShapeDtypeStruct((N, EMB), jnp.float32)] * 2,
    )(x, emb_W, emb_b.reshape(1, -1), Wq)


def _tc_edge_pre(ea, Wme, bm, Wce, bc):
    W = ea.shape[1]

    def body(ea_r, Wm_r, bm_r, Wc_r, bc_r, r_r, c_r):
        a = ea_r[:, :EDGE]
        r_r[...] = jnp.dot(a, Wm_r[...],
                           preferred_element_type=jnp.float32) + bm_r[...]
        c_r[...] = jnp.dot(a, Wc_r[...],
                           preferred_element_type=jnp.float32) + bc_r[...]

    return pl.pallas_call(
        body,
        grid=(E // EB,),
        in_specs=[_rows((EB, W)), _whole((EDGE, EMB)), _whole((1, EMB)),
                  _whole((EDGE, EDGE)), _whole((1, EDGE))],
        out_specs=[_rows((EB, EMB)), _rows((EB, EDGE))],
        out_shape=[jax.ShapeDtypeStruct((E, EMB), jnp.float32),
                   jax.ShapeDtypeStruct((E, EDGE), jnp.float32)],
    )(ea, Wme, bm.reshape(1, -1), Wce, bc.reshape(1, -1))


def _tc_node_update(aggr, h, We_a, We_h, be, Wq, Wts, Wtd):
    def body(a0_r, a1_r, h_r, Wa_r, Wh_r, b_r, Wq_r, Ws_r, Wd_r,
             hn_r, q_r, Ts_r, Td_r):
        a = a0_r[...] + a1_r[...]
        hn = jnp.maximum(
            jnp.dot(a, Wa_r[...], preferred_element_type=jnp.float32)
            + jnp.dot(h_r[...], Wh_r[...], preferred_element_type=jnp.float32)
            + b_r[...], 0.0)
        hn_r[...] = hn
        q_r[...] = jnp.dot(hn, Wq_r[...], preferred_element_type=jnp.float32)
        Ts_r[...] = jnp.dot(hn, Ws_r[...], preferred_element_type=jnp.float32)
        Td_r[...] = jnp.dot(hn, Wd_r[...], preferred_element_type=jnp.float32)

    return pl.pallas_call(
        body,
        grid=(N // NB,),
        in_specs=[_rows((NB, EMB)), _rows((NB, EMB)), _rows((NB, EMB)),
                  _whole((EMB, EMB)), _whole((EMB, EMB)), _whole((1, EMB)),
                  _whole((EMB, EMB)), _whole((EMB, EMB)), _whole((EMB, EMB))],
        out_specs=[_rows((NB, EMB)), _rows((NB, EMB)),
                   _rows((NB, EMB)), _rows((NB, EMB))],
        out_shape=[jax.ShapeDtypeStruct((N, EMB), jnp.float32)] * 4,
    )(aggr[0], aggr[1], h, We_a, We_h, be.reshape(1, -1), Wq, Wts, Wtd)


def _tc_node_final(aggr, h, We_a, We_h, be, node_W, node_b, Wts, Wtd):
    def body(a0_r, a1_r, h_r, Wa_r, Wh_r, b_r, Wn_r, bn_r, Ws_r, Wd_r,
             np_r, Ts_r, Td_r):
        a = a0_r[...] + a1_r[...]
        h2 = jnp.maximum(
            jnp.dot(a, Wa_r[...], preferred_element_type=jnp.float32)
            + jnp.dot(h_r[...], Wh_r[...], preferred_element_type=jnp.float32)
            + b_r[...], 0.0)
        np_r[...] = jnp.dot(h2, Wn_r[...],
                            preferred_element_type=jnp.float32) + bn_r[...]
        Ts_r[...] = jnp.dot(h2, Ws_r[...], preferred_element_type=jnp.float32)
        Td_r[...] = jnp.dot(h2, Wd_r[...], preferred_element_type=jnp.float32)

    return pl.pallas_call(
        body,
        grid=(N // NB,),
        in_specs=[_rows((NB, EMB)), _rows((NB, EMB)), _rows((NB, EMB)),
                  _whole((EMB, EMB)), _whole((EMB, EMB)), _whole((1, EMB)),
                  _whole((EMB, OUT)), _whole((1, OUT)),
                  _whole((EMB, EMB)), _whole((EMB, EMB))],
        out_specs=[_rows((NB, OUT)), _rows((NB, EMB)), _rows((NB, EMB))],
        out_shape=[jax.ShapeDtypeStruct((N, OUT), jnp.float32),
                   jax.ShapeDtypeStruct((N, EMB), jnp.float32),
                   jax.ShapeDtypeStruct((N, EMB), jnp.float32)],
    )(aggr[0], aggr[1], h, We_a, We_h, be.reshape(1, -1),
      node_W, node_b.reshape(1, -1), Wts, Wtd)


def _tc_edge_final(ea2e, ep_We, ep_b):
    def body(ea_r, Wp_r, bp_r, ep_r):
        ep_r[...] = (jnp.dot(ea_r[:, :EDGE], Wp_r[...],
                             preferred_element_type=jnp.float32)
                     + ea_r[:, EDGE:EDGE + 1] + bp_r[...])

    return pl.pallas_call(
        body,
        grid=(E // EB,),
        in_specs=[_rows((EB, 2 * EDGE)), _whole((EDGE, 1)), _whole((1, 1))],
        out_specs=_rows((EB, 1)),
        out_shape=jax.ShapeDtypeStruct((E, 1), jnp.float32),
    )(ea2e, ep_We, ep_b.reshape(1, 1))


def kernel(x, edge_attr, edge_index, emb_W, emb_b, Wm, bm, We, be,
           Weu, beu, node_W, node_b, ep_W, ep_b):
    src = edge_index[0]
    dst = edge_index[1]

    Wm_h = [Wm[i, :EMB] for i in range(2)]
    Wm_e = [Wm[i, EMB:] for i in range(2)]
    We_a = [We[i, :EMB] for i in range(2)]
    We_h = [We[i, EMB:] for i in range(2)]
    Weu_s = [Weu[i, :EMB] for i in range(2)]
    Weu_d = [Weu[i, EMB:2 * EMB] for i in range(2)]
    Weu_e = [Weu[i, 2 * EMB:] for i in range(2)]
    zcol = jnp.zeros((EMB, 1), jnp.float32)
    zpad = jnp.zeros((EMB, EMB - EDGE - 1), jnp.float32)
    Wts1 = jnp.concatenate([Weu_s[0], zcol, zpad], axis=1)
    Wtd1 = jnp.concatenate([Weu_d[0], zcol, zpad], axis=1)
    Wts2 = jnp.concatenate([Weu_s[1], ep_W[:EMB], zpad], axis=1)
    Wtd2 = jnp.concatenate([Weu_d[1], ep_W[EMB:2 * EMB], zpad], axis=1)
    ep_We = ep_W[2 * EMB:]

    h0, q0 = _tc_node0(x, emb_W, emb_b, Wm_h[0])
    r0, c0 = _tc_edge_pre(edge_attr, Wm_e[0], bm[0], Weu_e[0], beu[0])
    aggr0 = _sc_msg_aggr(q0, r0, src, dst)
    h1, q1, T1s, T1d = _tc_node_update(aggr0, h0, We_a[0], We_h[0], be[0],
                                       Wm_h[1], Wts1, Wtd1)
    ea1e = _sc_edge_update(T1s, T1d, c0, src, dst)

    r1, c1 = _tc_edge_pre(ea1e, Wm_e[1], bm[1], Weu_e[1], beu[1])
    aggr1 = _sc_msg_aggr(q1, r1, src, dst)
    node_pred, T2s, T2d = _tc_node_final(aggr1, h1, We_a[1], We_h[1], be[1],
                                         node_W, node_b, Wts2, Wtd2)
    ea2e = _sc_edge_update(T2s, T2d, c1, src, dst)
    edge_pred = _tc_edge_final(ea2e, ep_We, ep_b)

    return (node_pred, edge_pred)


# speedup vs baseline: 4.1241x; 1.3050x over previous
import functools

import jax
import jax.numpy as jnp
from jax import lax
from jax.experimental import pallas as pl
from jax.experimental.pallas import tpu as pltpu
from jax.experimental.pallas import tpu_sc as plsc

N = 10000
E = 320000
IN = 128
EMB = 128
EDGE = 16
OUT = 128

NC, NS = 2, 16
NW = NC * NS
EPW = E // NW
C = 80
NCH = EPW // C
NPAD = 10240
ROWS_PT = NPAD // NS
LN = 16

_mesh = plsc.VectorSubcoreMesh(
    core_axis_name="c", subcore_axis_name="s", num_cores=NC, num_subcores=NS
)


@functools.partial(
    pl.kernel,
    out_type=jax.ShapeDtypeStruct((NC, NPAD, EMB), jnp.float32),
    mesh=_mesh,
    scratch_types=[
        pltpu.VMEM((C,), jnp.int32),
        pltpu.VMEM((C,), jnp.int32),
        pltpu.VMEM((C,), jnp.int32),
        pltpu.VMEM((C,), jnp.int32),
        pltpu.VMEM((C,), jnp.int32),
        pltpu.VMEM((C,), jnp.int32),
        pltpu.VMEM((C, EMB), jnp.float32),
        pltpu.VMEM((C, EMB), jnp.float32),
        pltpu.VMEM((C, EMB), jnp.float32),
        pltpu.VMEM((C, EMB), jnp.float32),
        pltpu.VMEM_SHARED((NPAD, EMB), jnp.float32),
        pltpu.SemaphoreType.DMA,
        pltpu.SemaphoreType.DMA,
        pltpu.SemaphoreType.DMA,
        pltpu.SemaphoreType.DMA,
        pltpu.SemaphoreType.DMA,
        pltpu.SemaphoreType.DMA,
        pltpu.SemaphoreType.DMA,
        pltpu.SemaphoreType.DMA,
    ],
)
def _sc_msg_aggr(q_hbm, rc_hbm, src_hbm, dst_hbm, out_hbm,
                 ids0, ids1, idd0, idd1, idd2, idd3,
                 gbuf0, gbuf1, rbuf0, rbuf1, acc,
                 si0, si1, sg0, sg1, sr0, sr1, ss0, ss1):
    c = lax.axis_index("c")
    s = lax.axis_index("s")
    wid = c * NS + s
    ebase = wid * EPW
    row0 = s * ROWS_PT
    ids, idds = (ids0, ids1), (idd0, idd1, idd2, idd3)
    gbufs, rbufs = (gbuf0, gbuf1), (rbuf0, rbuf1)
    sis, sgs, srs = (si0, si1), (sg0, sg1), (sr0, sr1)
    sss = (ss0, ss1)

    def _issue_idx(k, b, bank):
        pltpu.async_copy(src_hbm.at[pl.ds(ebase + k * C, C)], ids[b], sis[b])
        pltpu.async_copy(dst_hbm.at[pl.ds(ebase + k * C, C)], idds[bank],
                         sis[b])

    def _wait_idx(k, b, bank):
        pltpu.make_async_copy(src_hbm.at[pl.ds(ebase + k * C, C)], ids[b],
                              sis[b]).wait()
        pltpu.make_async_copy(dst_hbm.at[pl.ds(ebase + k * C, C)],
                              idds[bank], sis[b]).wait()

    def _issue_data(k, b):
        pltpu.async_copy(q_hbm.at[ids[b]], gbufs[b], sgs[b])
        pltpu.async_copy(rc_hbm.at[pl.ds(ebase + k * C, C), pl.ds(0, EMB)],
                         rbufs[b], srs[b])

    _issue_idx(0, 0, 0)
    _issue_idx(1, 1, 1)

    zero = jnp.zeros((LN,), jnp.float32)

    def zfill(i, _):
        e = i // (EMB // LN)
        j = (i % (EMB // LN)) * LN
        gbuf0[e, pl.ds(j, LN)] = zero
        return 0

    lax.fori_loop(0, C * (EMB // LN), zfill, 0, unroll=4)

    def zissue(k, _):
        pltpu.async_copy(gbuf0, acc.at[pl.ds(row0 + k * C, C)], sg0)
        return 0

    lax.fori_loop(0, ROWS_PT // C, zissue, 0)

    def zdrain(k, _):
        pltpu.make_async_copy(gbuf0, acc.at[pl.ds(row0 + k * C, C)],
                              sg0).wait()
        return 0

    lax.fori_loop(0, ROWS_PT // C, zdrain, 0)
    _wait_idx(0, 0, 0)
    _issue_data(0, 0)
    plsc.subcore_barrier()

    def _step(k, b, bank, first=False):
        if not first:
            pltpu.make_async_copy(gbufs[1 - b], acc.at[idds[(bank - 1) % 4]],
                                  sss[1 - b]).wait()

        @pl.when(k + 1 < NCH)
        def _():
            _wait_idx(k + 1, 1 - b, (bank + 1) % 4)
            _issue_data(k + 1, 1 - b)

        pltpu.make_async_copy(q_hbm.at[ids[b]], gbufs[b], sgs[b]).wait()
        pltpu.make_async_copy(rc_hbm.at[pl.ds(ebase + k * C, C), pl.ds(0, EMB)],
                              rbufs[b], srs[b]).wait()
        g, r = gbufs[b], rbufs[b]

        @plsc.parallel_loop(0, C, 1, unroll=4)
        def cw(e):
            for j in range(EMB // LN):
                g[e, pl.ds(j * LN, LN)] = jnp.maximum(
                    g[e, pl.ds(j * LN, LN)] + r[e, pl.ds(j * LN, LN)], 0.0)

        pltpu.async_copy(g, acc.at[idds[bank]], sss[b], add=True)

        @pl.when(k + 2 < NCH)
        def _():
            _issue_idx(k + 2, b, (bank + 2) % 4)

    _step(jnp.int32(0), 0, 0, first=True)

    def body(k4, _):
        for t in range(4):
            k = 1 + 4 * k4 + t
            _step(k, (1 + t) % 2, (1 + t) % 4)
        return 0

    lax.fori_loop(0, (NCH - 1) // 4, body, 0)
    pltpu.make_async_copy(gbufs[0], acc.at[idds[0]], sss[0]).wait()
    plsc.subcore_barrier()

    def ocopy(k2, _):
        for t in range(2):
            k = 2 * k2 + t
            g = gbufs[t]

            @pl.when(k >= 2)
            def _():
                pltpu.make_async_copy(
                    g, out_hbm.at[c, pl.ds(row0 + (k - 2) * C, C)],
                    sgs[t]).wait()

            pltpu.sync_copy(acc.at[pl.ds(row0 + k * C, C)], g)
            pltpu.async_copy(g, out_hbm.at[c, pl.ds(row0 + k * C, C)],
                             sgs[t])
        return 0

    lax.fori_loop(0, ROWS_PT // (2 * C), ocopy, 0)
    for t in range(2):
        k = ROWS_PT // C - 2 + t
        pltpu.make_async_copy(gbufs[t],
                              out_hbm.at[c, pl.ds(row0 + k * C, C)],
                              sgs[t]).wait()


@functools.partial(
    pl.kernel,
    out_type=jax.ShapeDtypeStruct((E, 2 * EDGE), jnp.float32),
    mesh=_mesh,
    scratch_types=[
        pltpu.VMEM((EPW,), jnp.int32),
        pltpu.VMEM((EPW,), jnp.int32),
        pltpu.VMEM((C, EMB), jnp.float32),
        pltpu.VMEM((C, EMB), jnp.float32),
        pltpu.VMEM((C, EMB), jnp.float32),
        pltpu.VMEM((C, EMB), jnp.float32),
        pltpu.VMEM((C, EDGE), jnp.float32),
        pltpu.VMEM((C, EDGE), jnp.float32),
        pltpu.VMEM((C, 2 * EDGE), jnp.float32),
        pltpu.VMEM((C, 2 * EDGE), jnp.float32),
        pltpu.SemaphoreType.DMA,
        pltpu.SemaphoreType.DMA,
        pltpu.SemaphoreType.DMA,
        pltpu.SemaphoreType.DMA,
        pltpu.SemaphoreType.DMA,
        pltpu.SemaphoreType.DMA,
        pltpu.SemaphoreType.DMA,
        pltpu.SemaphoreType.DMA,
    ],
)
def _sc_edge_update(ts_hbm, td_hbm, rc_hbm, src_hbm, dst_hbm, ea_hbm,
                    idx_s, idx_d, sbuf0, sbuf1, dbuf0, dbuf1,
                    cbuf0, cbuf1, eabuf0, eabuf1,
                    ss0, ss1, sd0, sd1, sc0, sc1, so0, so1):
    cc = lax.axis_index("c")
    s = lax.axis_index("s")
    wid = cc * NS + s
    ebase = wid * EPW
    sbufs, dbufs = (sbuf0, sbuf1), (dbuf0, dbuf1)
    cbufs, eabufs = (cbuf0, cbuf1), (eabuf0, eabuf1)
    sss, sds, scs = (ss0, ss1), (sd0, sd1), (sc0, sc1)
    sos = (so0, so1)

    pltpu.sync_copy(src_hbm.at[pl.ds(ebase, EPW)], idx_s)
    pltpu.sync_copy(dst_hbm.at[pl.ds(ebase, EPW)], idx_d)

    def _issue(k, b):
        pltpu.async_copy(ts_hbm.at[idx_s.at[pl.ds(k * C, C)]], sbufs[b],
                         sss[b])
        pltpu.async_copy(td_hbm.at[idx_d.at[pl.ds(k * C, C)]], dbufs[b],
                         sds[b])
        pltpu.async_copy(rc_hbm.at[pl.ds(ebase + k * C, C), pl.ds(EMB, EDGE)],
                         cbufs[b], scs[b])

    def _step(k, b, tail=False):
        if not tail:
            _issue(k + 1, 1 - b)

        pltpu.make_async_copy(ts_hbm.at[idx_s.at[pl.ds(k * C, C)]], sbufs[b],
                              sss[b]).wait()
        pltpu.make_async_copy(td_hbm.at[idx_d.at[pl.ds(k * C, C)]], dbufs[b],
                              sds[b]).wait()
        pltpu.make_async_copy(rc_hbm.at[pl.ds(ebase + k * C, C), pl.ds(EMB, EDGE)],
                              cbufs[b], scs[b]).wait()
        sb, db, cb, eb = sbufs[b], dbufs[b], cbufs[b], eabufs[b]

        @pl.when(k >= 2)
        def _():
            pltpu.make_async_copy(
                eb, ea_hbm.at[pl.ds(ebase + (k - 2) * C, C)], sos[b]).wait()

        @plsc.parallel_loop(0, C, 1, unroll=4)
        def cw(e):
            eb[e, pl.ds(0, LN)] = jnp.maximum(
                sb[e, pl.ds(0, LN)] + db[e, pl.ds(0, LN)]
                + cb[e, pl.ds(0, LN)], 0.0)
            eb[e, pl.ds(LN, LN)] = (sb[e, pl.ds(LN, LN)]
                                    + db[e, pl.ds(LN, LN)])
        pltpu.async_copy(eb, ea_hbm.at[pl.ds(ebase + k * C, C)], sos[b])

    _issue(0, 0)

    def body(k2, _):
        _step(2 * k2, 0)
        _step(2 * k2 + 1, 1)
        return 0

    lax.fori_loop(0, NCH // 2, body, 0)
    _step(jnp.int32(NCH - 1), (NCH - 1) % 2, tail=True)
    pltpu.make_async_copy(
        eabufs[(NCH - 2) % 2],
        ea_hbm.at[pl.ds(ebase + (NCH - 2) * C, C)], sos[(NCH - 2) % 2]).wait()
    pltpu.make_async_copy(
        eabufs[(NCH - 1) % 2],
        ea_hbm.at[pl.ds(ebase + (NCH - 1) * C, C)], sos[(NCH - 1) % 2]).wait()


NB = 1000
EB = 3200


def _rows(bshape):
    return pl.BlockSpec(bshape, lambda i: (i,) + (0,) * (len(bshape) - 1))


def _whole(shape):
    return pl.BlockSpec(shape, lambda i: (0,) * len(shape))


def _tc_node0(x, emb_W, emb_b, Wq):
    def body(x_r, W_r, b_r, Wq_r, h_r, q_r):
        h = jnp.dot(x_r[...], W_r[...], preferred_element_type=jnp.float32) + b_r[...]
        h_r[...] = h
        q_r[...] = jnp.dot(h, Wq_r[...], preferred_element_type=jnp.float32)

    return pl.pallas_call(
        body,
        grid=(N // NB,),
        in_specs=[_rows((NB, IN)), _whole((IN, EMB)), _whole((1, EMB)),
                  _whole((EMB, EMB))],
        out_specs=[_rows((NB, EMB)), _rows((NB, EMB))],
        out_shape=[jax.ShapeDtypeStruct((N, EMB), jnp.float32)] * 2,
    )(x, emb_W, emb_b.reshape(1, -1), Wq)


def _tc_edge_pre_t(eaT, Wme, bm, Wce, bc):
    dn = (((0,), (0,)), ((), ()))

    def body(ea_r, Wm_r, bm_r, Wc_r, bc_r, rc_r):
        a = ea_r[...]
        rc_r[:, :EMB] = lax.dot_general(
            a, Wm_r[...], dn, preferred_element_type=jnp.float32) + bm_r[...]
        rc_r[:, EMB:] = lax.dot_general(
            a, Wc_r[...], dn, preferred_element_type=jnp.float32) + bc_r[...]

    return pl.pallas_call(
        body,
        grid=(E // EB,),
        in_specs=[pl.BlockSpec((EDGE, EB), lambda i: (0, i)),
                  _whole((EDGE, EMB)), _whole((1, EMB)),
                  _whole((EDGE, EDGE)), _whole((1, EDGE))],
        out_specs=_rows((EB, EMB + EDGE)),
        out_shape=jax.ShapeDtypeStruct((E, EMB + EDGE), jnp.float32),
    )(eaT, Wme, bm.reshape(1, -1), Wce, bc.reshape(1, -1))


def _tc_edge_pre(ea, Wme, bm, Wce, bc):
    W = ea.shape[1]

    def body(ea_r, Wm_r, bm_r, Wc_r, bc_r, rc_r):
        a = ea_r[:, :EDGE]
        rc_r[:, :EMB] = jnp.dot(a, Wm_r[...],
                                preferred_element_type=jnp.float32) + bm_r[...]
        rc_r[:, EMB:] = jnp.dot(a, Wc_r[...],
                                preferred_element_type=jnp.float32) + bc_r[...]

    return pl.pallas_call(
        body,
        grid=(E // EB,),
        in_specs=[_rows((EB, W)), _whole((EDGE, EMB)), _whole((1, EMB)),
                  _whole((EDGE, EDGE)), _whole((1, EDGE))],
        out_specs=_rows((EB, EMB + EDGE)),
        out_shape=jax.ShapeDtypeStruct((E, EMB + EDGE), jnp.float32),
    )(ea, Wme, bm.reshape(1, -1), Wce, bc.reshape(1, -1))


def _tc_node_update(aggr, h, We_a, We_h, be, Wq, Wts, Wtd):
    def body(a0_r, a1_r, h_r, Wa_r, Wh_r, b_r, Wq_r, Ws_r, Wd_r,
             hn_r, q_r, Ts_r, Td_r):
        a = a0_r[...] + a1_r[...]
        hn = jnp.maximum(
            jnp.dot(a, Wa_r[...], preferred_element_type=jnp.float32)
            + jnp.dot(h_r[...], Wh_r[...], preferred_element_type=jnp.float32)
            + b_r[...], 0.0)
        hn_r[...] = hn
        q_r[...] = jnp.dot(hn, Wq_r[...], preferred_element_type=jnp.float32)
        Ts_r[...] = jnp.dot(hn, Ws_r[...], preferred_element_type=jnp.float32)
        Td_r[...] = jnp.dot(hn, Wd_r[...], preferred_element_type=jnp.float32)

    return pl.pallas_call(
        body,
        grid=(N // NB,),
        in_specs=[_rows((NB, EMB)), _rows((NB, EMB)), _rows((NB, EMB)),
                  _whole((EMB, EMB)), _whole((EMB, EMB)), _whole((1, EMB)),
                  _whole((EMB, EMB)), _whole((EMB, EMB)), _whole((EMB, EMB))],
        out_specs=[_rows((NB, EMB)), _rows((NB, EMB)),
                   _rows((NB, EMB)), _rows((NB, EMB))],
        out_shape=[jax.ShapeDtypeStruct((N, EMB), jnp.float32)] * 4,
    )(aggr[0], aggr[1], h, We_a, We_h, be.reshape(1, -1), Wq, Wts, Wtd)


def _tc_node_final(aggr, h, We_a, We_h, be, node_W, node_b, Wts, Wtd):
    def body(a0_r, a1_r, h_r, Wa_r, Wh_r, b_r, Wn_r, bn_r, Ws_r, Wd_r,
             np_r, Ts_r, Td_r):
        a = a0_r[...] + a1_r[...]
        h2 = jnp.maximum(
            jnp.dot(a, Wa_r[...], preferred_element_type=jnp.float32)
            + jnp.dot(h_r[...], Wh_r[...], preferred_element_type=jnp.float32)
            + b_r[...], 0.0)
        np_r[...] = jnp.dot(h2, Wn_r[...],
                            preferred_element_type=jnp.float32) + bn_r[...]
        Ts_r[...] = jnp.dot(h2, Ws_r[...], preferred_element_type=jnp.float32)
        Td_r[...] = jnp.dot(h2, Wd_r[...], preferred_element_type=jnp.float32)

    return pl.pallas_call(
        body,
        grid=(N // NB,),
        in_specs=[_rows((NB, EMB)), _rows((NB, EMB)), _rows((NB, EMB)),
                  _whole((EMB, EMB)), _whole((EMB, EMB)), _whole((1, EMB)),
                  _whole((EMB, OUT)), _whole((1, OUT)),
                  _whole((EMB, EMB)), _whole((EMB, EMB))],
        out_specs=[_rows((NB, OUT)), _rows((NB, EMB)), _rows((NB, EMB))],
        out_shape=[jax.ShapeDtypeStruct((N, OUT), jnp.float32),
                   jax.ShapeDtypeStruct((N, EMB), jnp.float32),
                   jax.ShapeDtypeStruct((N, EMB), jnp.float32)],
    )(aggr[0], aggr[1], h, We_a, We_h, be.reshape(1, -1),
      node_W, node_b.reshape(1, -1), Wts, Wtd)


def _tc_edge_final(ea2e, ep_We, ep_b):
    w17 = jnp.concatenate([ep_We[:, 0], jnp.ones((1,), jnp.float32)])
    w17 = w17.reshape(1, EDGE + 1)
    dn = (((1,), (1,)), ((), ()))

    def body(ea_r, Wp_r, bp_r, ep_r):
        ep_r[...] = lax.dot_general(
            Wp_r[...], ea_r[:, :EDGE + 1], dn,
            preferred_element_type=jnp.float32) + bp_r[...]

    return pl.pallas_call(
        body,
        grid=(E // EB,),
        in_specs=[_rows((EB, 2 * EDGE)), _whole((1, EDGE + 1)),
                  _whole((1, 1))],
        out_specs=pl.BlockSpec((1, EB), lambda i: (0, i)),
        out_shape=jax.ShapeDtypeStruct((1, E), jnp.float32),
    )(ea2e, w17, ep_b.reshape(1, 1))


def kernel(x, edge_attr, edge_index, emb_W, emb_b, Wm, bm, We, be,
           Weu, beu, node_W, node_b, ep_W, ep_b):
    src = edge_index[0]
    dst = edge_index[1]

    Wm_h = [Wm[i, :EMB] for i in range(2)]
    Wm_e = [Wm[i, EMB:] for i in range(2)]
    We_a = [We[i, :EMB] for i in range(2)]
    We_h = [We[i, EMB:] for i in range(2)]
    Weu_s = [Weu[i, :EMB] for i in range(2)]
    Weu_d = [Weu[i, EMB:2 * EMB] for i in range(2)]
    Weu_e = [Weu[i, 2 * EMB:] for i in range(2)]
    zcol = jnp.zeros((EMB, 1), jnp.float32)
    zpad = jnp.zeros((EMB, EMB - EDGE - 1), jnp.float32)
    Wts1 = jnp.concatenate([Weu_s[0], zcol, zpad], axis=1)
    Wtd1 = jnp.concatenate([Weu_d[0], zcol, zpad], axis=1)
    Wts2 = jnp.concatenate([Weu_s[1], ep_W[:EMB], zpad], axis=1)
    Wtd2 = jnp.concatenate([Weu_d[1], ep_W[EMB:2 * EMB], zpad], axis=1)
    ep_We = ep_W[2 * EMB:]

    h0, q0 = _tc_node0(x, emb_W, emb_b, Wm_h[0])
    rc0 = _tc_edge_pre_t(edge_attr.T, Wm_e[0], bm[0], Weu_e[0], beu[0])
    aggr0 = _sc_msg_aggr(q0, rc0, src, dst)
    h1, q1, T1s, T1d = _tc_node_update(aggr0, h0, We_a[0], We_h[0], be[0],
                                       Wm_h[1], Wts1, Wtd1)
    ea1e = _sc_edge_update(T1s, T1d, rc0, src, dst)

    rc1 = _tc_edge_pre(ea1e, Wm_e[1], bm[1], Weu_e[1], beu[1])
    aggr1 = _sc_msg_aggr(q1, rc1, src, dst)
    node_pred, T2s, T2d = _tc_node_final(aggr1, h1, We_a[1], We_h[1], be[1],
                                         node_W, node_b, Wts2, Wtd2)
    ea2e = _sc_edge_update(T2s, T2d, rc1, src, dst)
    edge_pred = _tc_edge_final(ea2e, ep_We, ep_b).reshape(E, 1)

    return (node_pred, edge_pred)


# speedup vs baseline: 4.3402x; 1.0524x over previous
import functools

import jax
import jax.numpy as jnp
from jax import lax
from jax.experimental import pallas as pl
from jax.experimental.pallas import tpu as pltpu
from jax.experimental.pallas import tpu_sc as plsc

N = 10000
E = 320000
IN = 128
EMB = 128
EDGE = 16
OUT = 128

NC, NS = 2, 16
NW = NC * NS
EPW = E // NW
C = 80
NCH = EPW // C
NPAD = 10240
ROWS_PT = NPAD // NS
LN = 16

_mesh = plsc.VectorSubcoreMesh(
    core_axis_name="c", subcore_axis_name="s", num_cores=NC, num_subcores=NS
)


@functools.partial(
    pl.kernel,
    out_type=jax.ShapeDtypeStruct((NC, NPAD, EMB), jnp.float32),
    mesh=_mesh,
    scratch_types=[
        pltpu.VMEM((C,), jnp.int32),
        pltpu.VMEM((C,), jnp.int32),
        pltpu.VMEM((C,), jnp.int32),
        pltpu.VMEM((C,), jnp.int32),
        pltpu.VMEM((C,), jnp.int32),
        pltpu.VMEM((C,), jnp.int32),
        pltpu.VMEM((C, EMB), jnp.float32),
        pltpu.VMEM((C, EMB), jnp.float32),
        pltpu.VMEM((C, EMB), jnp.float32),
        pltpu.VMEM((C, EMB), jnp.float32),
        pltpu.VMEM_SHARED((NPAD, EMB), jnp.float32),
        pltpu.SemaphoreType.DMA,
        pltpu.SemaphoreType.DMA,
        pltpu.SemaphoreType.DMA,
        pltpu.SemaphoreType.DMA,
        pltpu.SemaphoreType.DMA,
        pltpu.SemaphoreType.DMA,
        pltpu.SemaphoreType.DMA,
        pltpu.SemaphoreType.DMA,
    ],
)
def _sc_msg_aggr(q_hbm, rc_hbm, src_hbm, dst_hbm, out_hbm,
                 ids0, ids1, idd0, idd1, idd2, idd3,
                 gbuf0, gbuf1, rbuf0, rbuf1, acc,
                 si0, si1, sg0, sg1, sr0, sr1, ss0, ss1):
    c = lax.axis_index("c")
    s = lax.axis_index("s")
    wid = c * NS + s
    ebase = wid * EPW
    row0 = s * ROWS_PT
    ids, idds = (ids0, ids1), (idd0, idd1, idd2, idd3)
    gbufs, rbufs = (gbuf0, gbuf1), (rbuf0, rbuf1)
    sis, sgs, srs = (si0, si1), (sg0, sg1), (sr0, sr1)
    sss = (ss0, ss1)

    def _issue_idx(k, b, bank):
        pltpu.async_copy(src_hbm.at[pl.ds(ebase + k * C, C)], ids[b], sis[b])
        pltpu.async_copy(dst_hbm.at[pl.ds(ebase + k * C, C)], idds[bank],
                         sis[b])

    def _wait_idx(k, b, bank):
        pltpu.make_async_copy(src_hbm.at[pl.ds(ebase + k * C, C)], ids[b],
                              sis[b]).wait()
        pltpu.make_async_copy(dst_hbm.at[pl.ds(ebase + k * C, C)],
                              idds[bank], sis[b]).wait()

    def _issue_data(k, b):
        pltpu.async_copy(q_hbm.at[ids[b]], gbufs[b], sgs[b])
        pltpu.async_copy(rc_hbm.at[pl.ds(ebase + k * C, C), pl.ds(0, EMB)],
                         rbufs[b], srs[b])

    _issue_idx(0, 0, 0)
    _issue_idx(1, 1, 1)

    zero = jnp.zeros((LN,), jnp.float32)

    def zfill(i, _):
        e = i // (EMB // LN)
        j = (i % (EMB // LN)) * LN
        gbuf0[e, pl.ds(j, LN)] = zero
        return 0

    lax.fori_loop(0, C * (EMB // LN), zfill, 0, unroll=4)

    def zissue(k, _):
        pltpu.async_copy(gbuf0, acc.at[pl.ds(row0 + k * C, C)], sg0)
        return 0

    lax.fori_loop(0, ROWS_PT // C, zissue, 0)

    def zdrain(k, _):
        pltpu.make_async_copy(gbuf0, acc.at[pl.ds(row0 + k * C, C)],
                              sg0).wait()
        return 0

    lax.fori_loop(0, ROWS_PT // C, zdrain, 0)
    _wait_idx(0, 0, 0)
    _issue_data(0, 0)
    plsc.subcore_barrier()

    def _step(k, b, bank, first=False):
        if not first:
            pltpu.make_async_copy(gbufs[1 - b], acc.at[idds[(bank - 1) % 4]],
                                  sss[1 - b]).wait()

        @pl.when(k + 1 < NCH)
        def _():
            _wait_idx(k + 1, 1 - b, (bank + 1) % 4)
            _issue_data(k + 1, 1 - b)

        pltpu.make_async_copy(q_hbm.at[ids[b]], gbufs[b], sgs[b]).wait()
        pltpu.make_async_copy(rc_hbm.at[pl.ds(ebase + k * C, C), pl.ds(0, EMB)],
                              rbufs[b], srs[b]).wait()
        g, r = gbufs[b], rbufs[b]

        @plsc.parallel_loop(0, C, 1, unroll=4)
        def cw(e):
            for j in range(EMB // LN):
                g[e, pl.ds(j * LN, LN)] = jnp.maximum(
                    g[e, pl.ds(j * LN, LN)] + r[e, pl.ds(j * LN, LN)], 0.0)

        pltpu.async_copy(g, acc.at[idds[bank]], sss[b], add=True)

        @pl.when(k + 2 < NCH)
        def _():
            _issue_idx(k + 2, b, (bank + 2) % 4)

    _step(jnp.int32(0), 0, 0, first=True)

    def body(k4, _):
        for t in range(4):
            k = 1 + 4 * k4 + t
            _step(k, (1 + t) % 2, (1 + t) % 4)
        return 0

    lax.fori_loop(0, (NCH - 1) // 4, body, 0)
    pltpu.make_async_copy(gbufs[0], acc.at[idds[0]], sss[0]).wait()
    plsc.subcore_barrier()

    def ocopy(k2, _):
        for t in range(2):
            k = 2 * k2 + t
            g = gbufs[t]

            @pl.when(k >= 2)
            def _():
                pltpu.make_async_copy(
                    g, out_hbm.at[c, pl.ds(row0 + (k - 2) * C, C)],
                    sgs[t]).wait()

            pltpu.sync_copy(acc.at[pl.ds(row0 + k * C, C)], g)
            pltpu.async_copy(g, out_hbm.at[c, pl.ds(row0 + k * C, C)],
                             sgs[t])
        return 0

    lax.fori_loop(0, ROWS_PT // (2 * C), ocopy, 0)
    for t in range(2):
        k = ROWS_PT // C - 2 + t
        pltpu.make_async_copy(gbufs[t],
                              out_hbm.at[c, pl.ds(row0 + k * C, C)],
                              sgs[t]).wait()


@functools.partial(
    pl.kernel,
    out_type=jax.ShapeDtypeStruct((E, 2 * EDGE), jnp.float32),
    mesh=_mesh,
    scratch_types=[
        pltpu.VMEM((EPW,), jnp.int32),
        pltpu.VMEM((EPW,), jnp.int32),
        pltpu.VMEM((C, EMB), jnp.float32),
        pltpu.VMEM((C, EMB), jnp.float32),
        pltpu.VMEM((C, EMB), jnp.float32),
        pltpu.VMEM((C, EMB), jnp.float32),
        pltpu.VMEM((C, EDGE), jnp.float32),
        pltpu.VMEM((C, EDGE), jnp.float32),
        pltpu.VMEM((C, 2 * EDGE), jnp.float32),
        pltpu.VMEM((C, 2 * EDGE), jnp.float32),
        pltpu.SemaphoreType.DMA,
        pltpu.SemaphoreType.DMA,
        pltpu.SemaphoreType.DMA,
        pltpu.SemaphoreType.DMA,
        pltpu.SemaphoreType.DMA,
        pltpu.SemaphoreType.DMA,
        pltpu.SemaphoreType.DMA,
        pltpu.SemaphoreType.DMA,
    ],
)
def _sc_edge_update(ts_hbm, td_hbm, rc_hbm, src_hbm, dst_hbm, ea_hbm,
                    idx_s, idx_d, sbuf0, sbuf1, dbuf0, dbuf1,
                    cbuf0, cbuf1, eabuf0, eabuf1,
                    ss0, ss1, sd0, sd1, sc0, sc1, so0, so1):
    cc = lax.axis_index("c")
    s = lax.axis_index("s")
    wid = cc * NS + s
    ebase = wid * EPW
    sbufs, dbufs = (sbuf0, sbuf1), (dbuf0, dbuf1)
    cbufs, eabufs = (cbuf0, cbuf1), (eabuf0, eabuf1)
    sss, sds, scs = (ss0, ss1), (sd0, sd1), (sc0, sc1)
    sos = (so0, so1)

    pltpu.sync_copy(src_hbm.at[pl.ds(ebase, EPW)], idx_s)
    pltpu.sync_copy(dst_hbm.at[pl.ds(ebase, EPW)], idx_d)

    def _issue(k, b):
        pltpu.async_copy(ts_hbm.at[idx_s.at[pl.ds(k * C, C)]], sbufs[b],
                         sss[b])
        pltpu.async_copy(td_hbm.at[idx_d.at[pl.ds(k * C, C)]], dbufs[b],
                         sds[b])
        pltpu.async_copy(rc_hbm.at[pl.ds(ebase + k * C, C), pl.ds(EMB, EDGE)],
                         cbufs[b], scs[b])

    def _step(k, b, tail=False):
        if not tail:
            _issue(k + 1, 1 - b)

        pltpu.make_async_copy(ts_hbm.at[idx_s.at[pl.ds(k * C, C)]], sbufs[b],
                              sss[b]).wait()
        pltpu.make_async_copy(td_hbm.at[idx_d.at[pl.ds(k * C, C)]], dbufs[b],
                              sds[b]).wait()
        pltpu.make_async_copy(rc_hbm.at[pl.ds(ebase + k * C, C), pl.ds(EMB, EDGE)],
                              cbufs[b], scs[b]).wait()
        sb, db, cb, eb = sbufs[b], dbufs[b], cbufs[b], eabufs[b]

        @pl.when(k >= 2)
        def _():
            pltpu.make_async_copy(
                eb, ea_hbm.at[pl.ds(ebase + (k - 2) * C, C)], sos[b]).wait()

        @plsc.parallel_loop(0, C, 1, unroll=4)
        def cw(e):
            eb[e, pl.ds(0, LN)] = jnp.maximum(
                sb[e, pl.ds(0, LN)] + db[e, pl.ds(0, LN)]
                + cb[e, pl.ds(0, LN)], 0.0)
            eb[e, pl.ds(LN, LN)] = (sb[e, pl.ds(LN, LN)]
                                    + db[e, pl.ds(LN, LN)])
        pltpu.async_copy(eb, ea_hbm.at[pl.ds(ebase + k * C, C)], sos[b])

    _issue(0, 0)

    def body(k2, _):
        _step(2 * k2, 0)
        _step(2 * k2 + 1, 1)
        return 0

    lax.fori_loop(0, NCH // 2, body, 0)
    _step(jnp.int32(NCH - 1), (NCH - 1) % 2, tail=True)
    pltpu.make_async_copy(
        eabufs[(NCH - 2) % 2],
        ea_hbm.at[pl.ds(ebase + (NCH - 2) * C, C)], sos[(NCH - 2) % 2]).wait()
    pltpu.make_async_copy(
        eabufs[(NCH - 1) % 2],
        ea_hbm.at[pl.ds(ebase + (NCH - 1) * C, C)], sos[(NCH - 1) % 2]).wait()


NB = 1000
EB = 6400


def _rows(bshape):
    return pl.BlockSpec(bshape, lambda i: (i,) + (0,) * (len(bshape) - 1))


def _whole(shape):
    return pl.BlockSpec(shape, lambda i: (0,) * len(shape))


def _tc_node0(x, emb_W, emb_b, Wq):
    def body(x_r, W_r, b_r, Wq_r, h_r, q_r):
        h = jnp.dot(x_r[...], W_r[...], preferred_element_type=jnp.float32) + b_r[...]
        h_r[...] = h
        q_r[...] = jnp.dot(h, Wq_r[...], preferred_element_type=jnp.float32)

    return pl.pallas_call(
        body,
        grid=(N // NB,),
        in_specs=[_rows((NB, IN)), _whole((IN, EMB)), _whole((1, EMB)),
                  _whole((EMB, EMB))],
        out_specs=[_rows((NB, EMB)), _rows((NB, EMB))],
        out_shape=[jax.ShapeDtypeStruct((N, EMB), jnp.float32)] * 2,
    )(x, emb_W, emb_b.reshape(1, -1), Wq)


def _tc_edge_pre_t(eaT, Wme, bm, Wce, bc):
    dn = (((0,), (0,)), ((), ()))

    def body(ea_r, Wm_r, bm_r, Wc_r, bc_r, rc_r):
        a = ea_r[...]
        rc_r[:, :EMB] = lax.dot_general(
            a, Wm_r[...], dn, preferred_element_type=jnp.float32) + bm_r[...]
        rc_r[:, EMB:] = lax.dot_general(
            a, Wc_r[...], dn, preferred_element_type=jnp.float32) + bc_r[...]

    return pl.pallas_call(
        body,
        grid=(E // EB,),
        in_specs=[pl.BlockSpec((EDGE, EB), lambda i: (0, i)),
                  _whole((EDGE, EMB)), _whole((1, EMB)),
                  _whole((EDGE, EDGE)), _whole((1, EDGE))],
        out_specs=_rows((EB, EMB + EDGE)),
        out_shape=jax.ShapeDtypeStruct((E, EMB + EDGE), jnp.float32),
    )(eaT, Wme, bm.reshape(1, -1), Wce, bc.reshape(1, -1))


def _tc_edge_pre(ea, Wme, bm, Wce, bc):
    W = ea.shape[1]

    def body(ea_r, Wm_r, bm_r, Wc_r, bc_r, rc_r):
        a = ea_r[:, :EDGE]
        rc_r[:, :EMB] = jnp.dot(a, Wm_r[...],
                                preferred_element_type=jnp.float32) + bm_r[...]
        rc_r[:, EMB:] = jnp.dot(a, Wc_r[...],
                                preferred_element_type=jnp.float32) + bc_r[...]

    return pl.pallas_call(
        body,
        grid=(E // EB,),
        in_specs=[_rows((EB, W)), _whole((EDGE, EMB)), _whole((1, EMB)),
                  _whole((EDGE, EDGE)), _whole((1, EDGE))],
        out_specs=_rows((EB, EMB + EDGE)),
        out_shape=jax.ShapeDtypeStruct((E, EMB + EDGE), jnp.float32),
    )(ea, Wme, bm.reshape(1, -1), Wce, bc.reshape(1, -1))


def _tc_node_update(aggr, h, We_a, We_h, be, Wq, Wts, Wtd):
    def body(a0_r, a1_r, h_r, Wa_r, Wh_r, b_r, Wq_r, Ws_r, Wd_r,
             hn_r, q_r, Ts_r, Td_r):
        a = a0_r[...] + a1_r[...]
        hn = jnp.maximum(
            jnp.dot(a, Wa_r[...], preferred_element_type=jnp.float32)
            + jnp.dot(h_r[...], Wh_r[...], preferred_element_type=jnp.float32)
            + b_r[...], 0.0)
        hn_r[...] = hn
        q_r[...] = jnp.dot(hn, Wq_r[...], preferred_element_type=jnp.float32)
        Ts_r[...] = jnp.dot(hn, Ws_r[...], preferred_element_type=jnp.float32)
        Td_r[...] = jnp.dot(hn, Wd_r[...], preferred_element_type=jnp.float32)

    return pl.pallas_call(
        body,
        grid=(N // NB,),
        in_specs=[_rows((NB, EMB)), _rows((NB, EMB)), _rows((NB, EMB)),
                  _whole((EMB, EMB)), _whole((EMB, EMB)), _whole((1, EMB)),
                  _whole((EMB, EMB)), _whole((EMB, EMB)), _whole((EMB, EMB))],
        out_specs=[_rows((NB, EMB)), _rows((NB, EMB)),
                   _rows((NB, EMB)), _rows((NB, EMB))],
        out_shape=[jax.ShapeDtypeStruct((N, EMB), jnp.float32)] * 4,
    )(aggr[0], aggr[1], h, We_a, We_h, be.reshape(1, -1), Wq, Wts, Wtd)


def _tc_node_final(aggr, h, We_a, We_h, be, node_W, node_b, Wts, Wtd):
    def body(a0_r, a1_r, h_r, Wa_r, Wh_r, b_r, Wn_r, bn_r, Ws_r, Wd_r,
             np_r, Ts_r, Td_r):
        a = a0_r[...] + a1_r[...]
        h2 = jnp.maximum(
            jnp.dot(a, Wa_r[...], preferred_element_type=jnp.float32)
            + jnp.dot(h_r[...], Wh_r[...], preferred_element_type=jnp.float32)
            + b_r[...], 0.0)
        np_r[...] = jnp.dot(h2, Wn_r[...],
                            preferred_element_type=jnp.float32) + bn_r[...]
        Ts_r[...] = jnp.dot(h2, Ws_r[...], preferred_element_type=jnp.float32)
        Td_r[...] = jnp.dot(h2, Wd_r[...], preferred_element_type=jnp.float32)

    return pl.pallas_call(
        body,
        grid=(N // NB,),
        in_specs=[_rows((NB, EMB)), _rows((NB, EMB)), _rows((NB, EMB)),
                  _whole((EMB, EMB)), _whole((EMB, EMB)), _whole((1, EMB)),
                  _whole((EMB, OUT)), _whole((1, OUT)),
                  _whole((EMB, EMB)), _whole((EMB, EMB))],
        out_specs=[_rows((NB, OUT)), _rows((NB, EMB)), _rows((NB, EMB))],
        out_shape=[jax.ShapeDtypeStruct((N, OUT), jnp.float32),
                   jax.ShapeDtypeStruct((N, EMB), jnp.float32),
                   jax.ShapeDtypeStruct((N, EMB), jnp.float32)],
    )(aggr[0], aggr[1], h, We_a, We_h, be.reshape(1, -1),
      node_W, node_b.reshape(1, -1), Wts, Wtd)


def _tc_edge_final(ea2e, ep_We, ep_b):
    w17 = jnp.concatenate([ep_We[:, 0], jnp.ones((1,), jnp.float32)])
    w17 = w17.reshape(1, EDGE + 1)
    dn = (((1,), (1,)), ((), ()))

    def body(ea_r, Wp_r, bp_r, ep_r):
        ep_r[...] = lax.dot_general(
            Wp_r[...], ea_r[:, :EDGE + 1], dn,
            preferred_element_type=jnp.float32) + bp_r[...]

    return pl.pallas_call(
        body,
        grid=(E // EB,),
        in_specs=[_rows((EB, 2 * EDGE)), _whole((1, EDGE + 1)),
                  _whole((1, 1))],
        out_specs=pl.BlockSpec((1, EB), lambda i: (0, i)),
        out_shape=jax.ShapeDtypeStruct((1, E), jnp.float32),
    )(ea2e, w17, ep_b.reshape(1, 1))


def kernel(x, edge_attr, edge_index, emb_W, emb_b, Wm, bm, We, be,
           Weu, beu, node_W, node_b, ep_W, ep_b):
    src = edge_index[0]
    dst = edge_index[1]

    Wm_h = [Wm[i, :EMB] for i in range(2)]
    Wm_e = [Wm[i, EMB:] for i in range(2)]
    We_a = [We[i, :EMB] for i in range(2)]
    We_h = [We[i, EMB:] for i in range(2)]
    Weu_s = [Weu[i, :EMB] for i in range(2)]
    Weu_d = [Weu[i, EMB:2 * EMB] for i in range(2)]
    Weu_e = [Weu[i, 2 * EMB:] for i in range(2)]
    zcol = jnp.zeros((EMB, 1), jnp.float32)
    zpad = jnp.zeros((EMB, EMB - EDGE - 1), jnp.float32)
    Wts1 = jnp.concatenate([Weu_s[0], zcol, zpad], axis=1)
    Wtd1 = jnp.concatenate([Weu_d[0], zcol, zpad], axis=1)
    Wts2 = jnp.concatenate([Weu_s[1], ep_W[:EMB], zpad], axis=1)
    Wtd2 = jnp.concatenate([Weu_d[1], ep_W[EMB:2 * EMB], zpad], axis=1)
    ep_We = ep_W[2 * EMB:]

    h0, q0 = _tc_node0(x, emb_W, emb_b, Wm_h[0])
    rc0 = _tc_edge_pre_t(edge_attr.T, Wm_e[0], bm[0], Weu_e[0], beu[0])
    aggr0 = _sc_msg_aggr(q0, rc0, src, dst)
    h1, q1, T1s, T1d = _tc_node_update(aggr0, h0, We_a[0], We_h[0], be[0],
                                       Wm_h[1], Wts1, Wtd1)
    ea1e = _sc_edge_update(T1s, T1d, rc0, src, dst)

    rc1 = _tc_edge_pre(ea1e, Wm_e[1], bm[1], Weu_e[1], beu[1])
    aggr1 = _sc_msg_aggr(q1, rc1, src, dst)
    node_pred, T2s, T2d = _tc_node_final(aggr1, h1, We_a[1], We_h[1], be[1],
                                         node_W, node_b, Wts2, Wtd2)
    ea2e = _sc_edge_update(T2s, T2d, rc1, src, dst)
    edge_pred = _tc_edge_final(ea2e, ep_We, ep_b).reshape(E, 1)

    return (node_pred, edge_pred)


# speedup vs baseline: 4.4159x; 1.0174x over previous
import functools

import jax
import jax.numpy as jnp
from jax import lax
from jax.experimental import pallas as pl
from jax.experimental.pallas import tpu as pltpu
from jax.experimental.pallas import tpu_sc as plsc

N = 10000
E = 320000
IN = 128
EMB = 128
EDGE = 16
OUT = 128

NC, NS = 2, 16
NW = NC * NS
EPW = E // NW
C = 80
NCH = EPW // C
NPAD = 10240
ROWS_PT = NPAD // NS
LN = 16

_mesh = plsc.VectorSubcoreMesh(
    core_axis_name="c", subcore_axis_name="s", num_cores=NC, num_subcores=NS
)


@functools.partial(
    pl.kernel,
    out_type=jax.ShapeDtypeStruct((NC, NPAD, EMB), jnp.float32),
    mesh=_mesh,
    scratch_types=[
        pltpu.VMEM((C,), jnp.int32),
        pltpu.VMEM((C,), jnp.int32),
        pltpu.VMEM((C,), jnp.int32),
        pltpu.VMEM((C,), jnp.int32),
        pltpu.VMEM((C,), jnp.int32),
        pltpu.VMEM((C,), jnp.int32),
        pltpu.VMEM((C, EMB), jnp.float32),
        pltpu.VMEM((C, EMB), jnp.float32),
        pltpu.VMEM((C, EMB), jnp.float32),
        pltpu.VMEM((C, EMB), jnp.float32),
        pltpu.VMEM_SHARED((NPAD, EMB), jnp.float32),
        pltpu.SemaphoreType.DMA,
        pltpu.SemaphoreType.DMA,
        pltpu.SemaphoreType.DMA,
        pltpu.SemaphoreType.DMA,
        pltpu.SemaphoreType.DMA,
        pltpu.SemaphoreType.DMA,
        pltpu.SemaphoreType.DMA,
        pltpu.SemaphoreType.DMA,
    ],
)
def _sc_msg_aggr(q_hbm, rc_hbm, src_hbm, dst_hbm, out_hbm,
                 ids0, ids1, idd0, idd1, idd2, idd3,
                 gbuf0, gbuf1, rbuf0, rbuf1, acc,
                 si0, si1, sg0, sg1, sr0, sr1, ss0, ss1):
    c = lax.axis_index("c")
    s = lax.axis_index("s")
    wid = c * NS + s
    ebase = wid * EPW
    row0 = s * ROWS_PT
    ids, idds = (ids0, ids1), (idd0, idd1, idd2, idd3)
    gbufs, rbufs = (gbuf0, gbuf1), (rbuf0, rbuf1)
    sis, sgs, srs = (si0, si1), (sg0, sg1), (sr0, sr1)
    sss = (ss0, ss1)

    def _issue_idx(k, b, bank):
        pltpu.async_copy(src_hbm.at[pl.ds(ebase + k * C, C)], ids[b], sis[b])
        pltpu.async_copy(dst_hbm.at[pl.ds(ebase + k * C, C)], idds[bank],
                         sis[b])

    def _wait_idx(k, b, bank):
        pltpu.make_async_copy(src_hbm.at[pl.ds(ebase + k * C, C)], ids[b],
                              sis[b]).wait()
        pltpu.make_async_copy(dst_hbm.at[pl.ds(ebase + k * C, C)],
                              idds[bank], sis[b]).wait()

    def _issue_data(k, b):
        pltpu.async_copy(q_hbm.at[ids[b]], gbufs[b], sgs[b])
        pltpu.async_copy(rc_hbm.at[pl.ds(ebase + k * C, C), pl.ds(0, EMB)],
                         rbufs[b], srs[b])

    _issue_idx(0, 0, 0)
    _issue_idx(1, 1, 1)

    zero = jnp.zeros((LN,), jnp.float32)

    def zfill(i, _):
        e = i // (EMB // LN)
        j = (i % (EMB // LN)) * LN
        gbuf0[e, pl.ds(j, LN)] = zero
        return 0

    lax.fori_loop(0, C * (EMB // LN), zfill, 0, unroll=4)

    def zissue(k, _):
        pltpu.async_copy(gbuf0, acc.at[pl.ds(row0 + k * C, C)], sg0)
        return 0

    lax.fori_loop(0, ROWS_PT // C, zissue, 0)

    def zdrain(k, _):
        pltpu.make_async_copy(gbuf0, acc.at[pl.ds(row0 + k * C, C)],
                              sg0).wait()
        return 0

    lax.fori_loop(0, ROWS_PT // C, zdrain, 0)
    _wait_idx(0, 0, 0)
    _issue_data(0, 0)
    plsc.subcore_barrier()

    def _step(k, b, bank, first=False):
        if not first:
            pltpu.make_async_copy(gbufs[1 - b], acc.at[idds[(bank - 1) % 4]],
                                  sss[1 - b]).wait()

        @pl.when(k + 1 < NCH)
        def _():
            _wait_idx(k + 1, 1 - b, (bank + 1) % 4)
            _issue_data(k + 1, 1 - b)

        pltpu.make_async_copy(q_hbm.at[ids[b]], gbufs[b], sgs[b]).wait()
        pltpu.make_async_copy(rc_hbm.at[pl.ds(ebase + k * C, C), pl.ds(0, EMB)],
                              rbufs[b], srs[b]).wait()
        g, r = gbufs[b], rbufs[b]

        @plsc.parallel_loop(0, C, 1, unroll=4)
        def cw(e):
            for j in range(EMB // LN):
                g[e, pl.ds(j * LN, LN)] = jnp.maximum(
                    g[e, pl.ds(j * LN, LN)] + r[e, pl.ds(j * LN, LN)], 0.0)

        pltpu.async_copy(g, acc.at[idds[bank]], sss[b], add=True)

        @pl.when(k + 2 < NCH)
        def _():
            _issue_idx(k + 2, b, (bank + 2) % 4)

    _step(jnp.int32(0), 0, 0, first=True)

    def body(k4, _):
        for t in range(4):
            k = 1 + 4 * k4 + t
            _step(k, (1 + t) % 2, (1 + t) % 4)
        return 0

    lax.fori_loop(0, (NCH - 1) // 4, body, 0)
    pltpu.make_async_copy(gbufs[0], acc.at[idds[0]], sss[0]).wait()
    plsc.subcore_barrier()

    def ocopy(k2, _):
        for t in range(2):
            k = 2 * k2 + t
            g = gbufs[t]

            @pl.when(k >= 2)
            def _():
                pltpu.make_async_copy(
                    g, out_hbm.at[c, pl.ds(row0 + (k - 2) * C, C)],
                    sgs[t]).wait()

            pltpu.sync_copy(acc.at[pl.ds(row0 + k * C, C)], g)
            pltpu.async_copy(g, out_hbm.at[c, pl.ds(row0 + k * C, C)],
                             sgs[t])
        return 0

    lax.fori_loop(0, ROWS_PT // (2 * C), ocopy, 0)
    for t in range(2):
        k = ROWS_PT // C - 2 + t
        pltpu.make_async_copy(gbufs[t],
                              out_hbm.at[c, pl.ds(row0 + k * C, C)],
                              sgs[t]).wait()


@functools.partial(
    pl.kernel,
    out_type=jax.ShapeDtypeStruct((E, 2 * EDGE), jnp.float32),
    mesh=_mesh,
    scratch_types=[
        pltpu.VMEM((EPW,), jnp.int32),
        pltpu.VMEM((EPW,), jnp.int32),
        pltpu.VMEM((C, EMB), jnp.float32),
        pltpu.VMEM((C, EMB), jnp.float32),
        pltpu.VMEM((C, EMB), jnp.float32),
        pltpu.VMEM((C, EMB), jnp.float32),
        pltpu.VMEM((C, EDGE), jnp.float32),
        pltpu.VMEM((C, EDGE), jnp.float32),
        pltpu.VMEM((C, 2 * EDGE), jnp.float32),
        pltpu.VMEM((C, 2 * EDGE), jnp.float32),
        pltpu.SemaphoreType.DMA,
        pltpu.SemaphoreType.DMA,
        pltpu.SemaphoreType.DMA,
        pltpu.SemaphoreType.DMA,
        pltpu.SemaphoreType.DMA,
        pltpu.SemaphoreType.DMA,
        pltpu.SemaphoreType.DMA,
        pltpu.SemaphoreType.DMA,
    ],
)
def _sc_edge_update(ts_hbm, td_hbm, rc_hbm, src_hbm, dst_hbm, ea_hbm,
                    idx_s, idx_d, sbuf0, sbuf1, dbuf0, dbuf1,
                    cbuf0, cbuf1, eabuf0, eabuf1,
                    ss0, ss1, sd0, sd1, sc0, sc1, so0, so1):
    cc = lax.axis_index("c")
    s = lax.axis_index("s")
    wid = cc * NS + s
    ebase = wid * EPW
    sbufs, dbufs = (sbuf0, sbuf1), (dbuf0, dbuf1)
    cbufs, eabufs = (cbuf0, cbuf1), (eabuf0, eabuf1)
    sss, sds, scs = (ss0, ss1), (sd0, sd1), (sc0, sc1)
    sos = (so0, so1)

    pltpu.sync_copy(src_hbm.at[pl.ds(ebase, EPW)], idx_s)
    pltpu.sync_copy(dst_hbm.at[pl.ds(ebase, EPW)], idx_d)

    def _issue(k, b):
        pltpu.async_copy(ts_hbm.at[idx_s.at[pl.ds(k * C, C)]], sbufs[b],
                         sss[b])
        pltpu.async_copy(td_hbm.at[idx_d.at[pl.ds(k * C, C)]], dbufs[b],
                         sds[b])
        pltpu.async_copy(rc_hbm.at[pl.ds(ebase + k * C, C), pl.ds(EMB, EDGE)],
                         cbufs[b], scs[b])

    def _step(k, b, tail=False):
        if not tail:
            _issue(k + 1, 1 - b)

        pltpu.make_async_copy(ts_hbm.at[idx_s.at[pl.ds(k * C, C)]], sbufs[b],
                              sss[b]).wait()
        pltpu.make_async_copy(td_hbm.at[idx_d.at[pl.ds(k * C, C)]], dbufs[b],
                              sds[b]).wait()
        pltpu.make_async_copy(rc_hbm.at[pl.ds(ebase + k * C, C), pl.ds(EMB, EDGE)],
                              cbufs[b], scs[b]).wait()
        sb, db, cb, eb = sbufs[b], dbufs[b], cbufs[b], eabufs[b]

        @pl.when(k >= 2)
        def _():
            pltpu.make_async_copy(
                eb, ea_hbm.at[pl.ds(ebase + (k - 2) * C, C)], sos[b]).wait()

        @plsc.parallel_loop(0, C, 1, unroll=4)
        def cw(e):
            eb[e, pl.ds(0, LN)] = jnp.maximum(
                sb[e, pl.ds(0, LN)] + db[e, pl.ds(0, LN)]
                + cb[e, pl.ds(0, LN)], 0.0)
            eb[e, pl.ds(LN, LN)] = (sb[e, pl.ds(LN, LN)]
                                    + db[e, pl.ds(LN, LN)])
        pltpu.async_copy(eb, ea_hbm.at[pl.ds(ebase + k * C, C)], sos[b])

    _issue(0, 0)

    def body(k2, _):
        _step(2 * k2, 0)
        _step(2 * k2 + 1, 1)
        return 0

    lax.fori_loop(0, NCH // 2, body, 0)
    _step(jnp.int32(NCH - 1), (NCH - 1) % 2, tail=True)
    pltpu.make_async_copy(
        eabufs[(NCH - 2) % 2],
        ea_hbm.at[pl.ds(ebase + (NCH - 2) * C, C)], sos[(NCH - 2) % 2]).wait()
    pltpu.make_async_copy(
        eabufs[(NCH - 1) % 2],
        ea_hbm.at[pl.ds(ebase + (NCH - 1) * C, C)], sos[(NCH - 1) % 2]).wait()


NB = 1000
EB = 12800


def _rows(bshape):
    return pl.BlockSpec(bshape, lambda i: (i,) + (0,) * (len(bshape) - 1))


def _whole(shape):
    return pl.BlockSpec(shape, lambda i: (0,) * len(shape))


def _tc_node0(x, emb_W, emb_b, Wq):
    def body(x_r, W_r, b_r, Wq_r, h_r, q_r):
        h = jnp.dot(x_r[...], W_r[...], preferred_element_type=jnp.float32) + b_r[...]
        h_r[...] = h
        q_r[...] = jnp.dot(h, Wq_r[...], preferred_element_type=jnp.float32)

    return pl.pallas_call(
        body,
        grid=(N // NB,),
        in_specs=[_rows((NB, IN)), _whole((IN, EMB)), _whole((1, EMB)),
                  _whole((EMB, EMB))],
        out_specs=[_rows((NB, EMB)), _rows((NB, EMB))],
        out_shape=[jax.ShapeDtypeStruct((N, EMB), jnp.float32)] * 2,
    )(x, emb_W, emb_b.reshape(1, -1), Wq)


def _tc_edge_pre_t(eaT, Wme, bm, Wce, bc):
    dn = (((0,), (0,)), ((), ()))

    def body(ea_r, Wm_r, bm_r, Wc_r, bc_r, rc_r):
        a = ea_r[...]
        rc_r[:, :EMB] = lax.dot_general(
            a, Wm_r[...], dn, preferred_element_type=jnp.float32) + bm_r[...]
        rc_r[:, EMB:] = lax.dot_general(
            a, Wc_r[...], dn, preferred_element_type=jnp.float32) + bc_r[...]

    return pl.pallas_call(
        body,
        grid=(E // EB,),
        in_specs=[pl.BlockSpec((EDGE, EB), lambda i: (0, i)),
                  _whole((EDGE, EMB)), _whole((1, EMB)),
                  _whole((EDGE, EDGE)), _whole((1, EDGE))],
        out_specs=_rows((EB, EMB + EDGE)),
        out_shape=jax.ShapeDtypeStruct((E, EMB + EDGE), jnp.float32),
    )(eaT, Wme, bm.reshape(1, -1), Wce, bc.reshape(1, -1))


def _tc_edge_pre(ea, Wme, bm, Wce, bc):
    W = ea.shape[1]

    def body(ea_r, Wm_r, bm_r, Wc_r, bc_r, rc_r):
        a = ea_r[:, :EDGE]
        rc_r[:, :EMB] = jnp.dot(a, Wm_r[...],
                                preferred_element_type=jnp.float32) + bm_r[...]
        rc_r[:, EMB:] = jnp.dot(a, Wc_r[...],
                                preferred_element_type=jnp.float32) + bc_r[...]

    return pl.pallas_call(
        body,
        grid=(E // EB,),
        in_specs=[_rows((EB, W)), _whole((EDGE, EMB)), _whole((1, EMB)),
                  _whole((EDGE, EDGE)), _whole((1, EDGE))],
        out_specs=_rows((EB, EMB + EDGE)),
        out_shape=jax.ShapeDtypeStruct((E, EMB + EDGE), jnp.float32),
    )(ea, Wme, bm.reshape(1, -1), Wce, bc.reshape(1, -1))


def _tc_node_update(aggr, h, We_a, We_h, be, Wq, Wts, Wtd):
    def body(a0_r, a1_r, h_r, Wa_r, Wh_r, b_r, Wq_r, Ws_r, Wd_r,
             hn_r, q_r, Ts_r, Td_r):
        a = a0_r[...] + a1_r[...]
        hn = jnp.maximum(
            jnp.dot(a, Wa_r[...], preferred_element_type=jnp.float32)
            + jnp.dot(h_r[...], Wh_r[...], preferred_element_type=jnp.float32)
            + b_r[...], 0.0)
        hn_r[...] = hn
        q_r[...] = jnp.dot(hn, Wq_r[...], preferred_element_type=jnp.float32)
        Ts_r[...] = jnp.dot(hn, Ws_r[...], preferred_element_type=jnp.float32)
        Td_r[...] = jnp.dot(hn, Wd_r[...], preferred_element_type=jnp.float32)

    return pl.pallas_call(
        body,
        grid=(N // NB,),
        in_specs=[_rows((NB, EMB)), _rows((NB, EMB)), _rows((NB, EMB)),
                  _whole((EMB, EMB)), _whole((EMB, EMB)), _whole((1, EMB)),
                  _whole((EMB, EMB)), _whole((EMB, EMB)), _whole((EMB, EMB))],
        out_specs=[_rows((NB, EMB)), _rows((NB, EMB)),
                   _rows((NB, EMB)), _rows((NB, EMB))],
        out_shape=[jax.ShapeDtypeStruct((N, EMB), jnp.float32)] * 4,
    )(aggr[0], aggr[1], h, We_a, We_h, be.reshape(1, -1), Wq, Wts, Wtd)


def _tc_node_final(aggr, h, We_a, We_h, be, node_W, node_b, Wts, Wtd):
    def body(a0_r, a1_r, h_r, Wa_r, Wh_r, b_r, Wn_r, bn_r, Ws_r, Wd_r,
             np_r, Ts_r, Td_r):
        a = a0_r[...] + a1_r[...]
        h2 = jnp.maximum(
            jnp.dot(a, Wa_r[...], preferred_element_type=jnp.float32)
            + jnp.dot(h_r[...], Wh_r[...], preferred_element_type=jnp.float32)
            + b_r[...], 0.0)
        np_r[...] = jnp.dot(h2, Wn_r[...],
                            preferred_element_type=jnp.float32) + bn_r[...]
        Ts_r[...] = jnp.dot(h2, Ws_r[...], preferred_element_type=jnp.float32)
        Td_r[...] = jnp.dot(h2, Wd_r[...], preferred_element_type=jnp.float32)

    return pl.pallas_call(
        body,
        grid=(N // NB,),
        in_specs=[_rows((NB, EMB)), _rows((NB, EMB)), _rows((NB, EMB)),
                  _whole((EMB, EMB)), _whole((EMB, EMB)), _whole((1, EMB)),
                  _whole((EMB, OUT)), _whole((1, OUT)),
                  _whole((EMB, EMB)), _whole((EMB, EMB))],
        out_specs=[_rows((NB, OUT)), _rows((NB, EMB)), _rows((NB, EMB))],
        out_shape=[jax.ShapeDtypeStruct((N, OUT), jnp.float32),
                   jax.ShapeDtypeStruct((N, EMB), jnp.float32),
                   jax.ShapeDtypeStruct((N, EMB), jnp.float32)],
    )(aggr[0], aggr[1], h, We_a, We_h, be.reshape(1, -1),
      node_W, node_b.reshape(1, -1), Wts, Wtd)


def _tc_edge_final(ea2e, ep_We, ep_b):
    w17 = jnp.concatenate([ep_We[:, 0], jnp.ones((1,), jnp.float32)])
    w17 = w17.reshape(1, EDGE + 1)
    dn = (((1,), (1,)), ((), ()))

    def body(ea_r, Wp_r, bp_r, ep_r):
        ep_r[...] = lax.dot_general(
            Wp_r[...], ea_r[:, :EDGE + 1], dn,
            preferred_element_type=jnp.float32) + bp_r[...]

    return pl.pallas_call(
        body,
        grid=(E // EB,),
        in_specs=[_rows((EB, 2 * EDGE)), _whole((1, EDGE + 1)),
                  _whole((1, 1))],
        out_specs=pl.BlockSpec((1, EB), lambda i: (0, i)),
        out_shape=jax.ShapeDtypeStruct((1, E), jnp.float32),
    )(ea2e, w17, ep_b.reshape(1, 1))


def kernel(x, edge_attr, edge_index, emb_W, emb_b, Wm, bm, We, be,
           Weu, beu, node_W, node_b, ep_W, ep_b):
    src = edge_index[0]
    dst = edge_index[1]

    Wm_h = [Wm[i, :EMB] for i in range(2)]
    Wm_e = [Wm[i, EMB:] for i in range(2)]
    We_a = [We[i, :EMB] for i in range(2)]
    We_h = [We[i, EMB:] for i in range(2)]
    Weu_s = [Weu[i, :EMB] for i in range(2)]
    Weu_d = [Weu[i, EMB:2 * EMB] for i in range(2)]
    Weu_e = [Weu[i, 2 * EMB:] for i in range(2)]
    zcol = jnp.zeros((EMB, 1), jnp.float32)
    zpad = jnp.zeros((EMB, EMB - EDGE - 1), jnp.float32)
    Wts1 = jnp.concatenate([Weu_s[0], zcol, zpad], axis=1)
    Wtd1 = jnp.concatenate([Weu_d[0], zcol, zpad], axis=1)
    Wts2 = jnp.concatenate([Weu_s[1], ep_W[:EMB], zpad], axis=1)
    Wtd2 = jnp.concatenate([Weu_d[1], ep_W[EMB:2 * EMB], zpad], axis=1)
    ep_We = ep_W[2 * EMB:]

    h0, q0 = _tc_node0(x, emb_W, emb_b, Wm_h[0])
    rc0 = _tc_edge_pre_t(edge_attr.T, Wm_e[0], bm[0], Weu_e[0], beu[0])
    aggr0 = _sc_msg_aggr(q0, rc0, src, dst)
    h1, q1, T1s, T1d = _tc_node_update(aggr0, h0, We_a[0], We_h[0], be[0],
                                       Wm_h[1], Wts1, Wtd1)
    ea1e = _sc_edge_update(T1s, T1d, rc0, src, dst)

    rc1 = _tc_edge_pre(ea1e, Wm_e[1], bm[1], Weu_e[1], beu[1])
    aggr1 = _sc_msg_aggr(q1, rc1, src, dst)
    node_pred, T2s, T2d = _tc_node_final(aggr1, h1, We_a[1], We_h[1], be[1],
                                         node_W, node_b, Wts2, Wtd2)
    ea2e = _sc_edge_update(T2s, T2d, rc1, src, dst)
    edge_pred = _tc_edge_final(ea2e, ep_We, ep_b).reshape(E, 1)

    return (node_pred, edge_pred)


# speedup vs baseline: 4.4228x; 1.0016x over previous
import functools

import jax
import jax.numpy as jnp
from jax import lax
from jax.experimental import pallas as pl
from jax.experimental.pallas import tpu as pltpu
from jax.experimental.pallas import tpu_sc as plsc

N = 10000
E = 320000
IN = 128
EMB = 128
EDGE = 16
OUT = 128

NC, NS = 2, 16
NW = NC * NS
EPW = E // NW
C = 80
NCH = EPW // C
NPAD = 10240
ROWS_PT = NPAD // NS
LN = 16

_mesh = plsc.VectorSubcoreMesh(
    core_axis_name="c", subcore_axis_name="s", num_cores=NC, num_subcores=NS
)


@functools.partial(
    pl.kernel,
    out_type=jax.ShapeDtypeStruct((NC, NPAD, EMB), jnp.float32),
    mesh=_mesh,
    scratch_types=[
        pltpu.VMEM((C,), jnp.int32),
        pltpu.VMEM((C,), jnp.int32),
        pltpu.VMEM((C,), jnp.int32),
        pltpu.VMEM((C,), jnp.int32),
        pltpu.VMEM((C,), jnp.int32),
        pltpu.VMEM((C,), jnp.int32),
        pltpu.VMEM((C, EMB), jnp.float32),
        pltpu.VMEM((C, EMB), jnp.float32),
        pltpu.VMEM((C, EMB), jnp.float32),
        pltpu.VMEM((C, EMB), jnp.float32),
        pltpu.VMEM_SHARED((NPAD, EMB), jnp.float32),
        pltpu.SemaphoreType.DMA,
        pltpu.SemaphoreType.DMA,
        pltpu.SemaphoreType.DMA,
        pltpu.SemaphoreType.DMA,
        pltpu.SemaphoreType.DMA,
        pltpu.SemaphoreType.DMA,
        pltpu.SemaphoreType.DMA,
        pltpu.SemaphoreType.DMA,
    ],
)
def _sc_msg_aggr(q_hbm, rc_hbm, src_hbm, dst_hbm, out_hbm,
                 ids0, ids1, idd0, idd1, idd2, idd3,
                 gbuf0, gbuf1, rbuf0, rbuf1, acc,
                 si0, si1, sg0, sg1, sr0, sr1, ss0, ss1):
    c = lax.axis_index("c")
    s = lax.axis_index("s")
    wid = c * NS + s
    ebase = wid * EPW
    row0 = s * ROWS_PT
    ids, idds = (ids0, ids1), (idd0, idd1, idd2, idd3)
    gbufs, rbufs = (gbuf0, gbuf1), (rbuf0, rbuf1)
    sis, sgs, srs = (si0, si1), (sg0, sg1), (sr0, sr1)
    sss = (ss0, ss1)

    def _issue_idx(k, b, bank):
        pltpu.async_copy(src_hbm.at[pl.ds(ebase + k * C, C)], ids[b], sis[b])
        pltpu.async_copy(dst_hbm.at[pl.ds(ebase + k * C, C)], idds[bank],
                         sis[b])

    def _wait_idx(k, b, bank):
        pltpu.make_async_copy(src_hbm.at[pl.ds(ebase + k * C, C)], ids[b],
                              sis[b]).wait()
        pltpu.make_async_copy(dst_hbm.at[pl.ds(ebase + k * C, C)],
                              idds[bank], sis[b]).wait()

    def _issue_data(k, b):
        pltpu.async_copy(q_hbm.at[ids[b]], gbufs[b], sgs[b])
        pltpu.async_copy(rc_hbm.at[pl.ds(ebase + k * C, C), pl.ds(0, EMB)],
                         rbufs[b], srs[b])

    _issue_idx(0, 0, 0)
    _issue_idx(1, 1, 1)

    zero = jnp.zeros((LN,), jnp.float32)

    def zfill(i, _):
        e = i // (EMB // LN)
        j = (i % (EMB // LN)) * LN
        gbuf0[e, pl.ds(j, LN)] = zero
        return 0

    lax.fori_loop(0, C * (EMB // LN), zfill, 0, unroll=4)

    def zissue(k, _):
        pltpu.async_copy(gbuf0, acc.at[pl.ds(row0 + k * C, C)], sg0)
        return 0

    lax.fori_loop(0, ROWS_PT // C, zissue, 0)

    def zdrain(k, _):
        pltpu.make_async_copy(gbuf0, acc.at[pl.ds(row0 + k * C, C)],
                              sg0).wait()
        return 0

    lax.fori_loop(0, ROWS_PT // C, zdrain, 0)
    _wait_idx(0, 0, 0)
    _issue_data(0, 0)
    plsc.subcore_barrier()

    def _step(k, b, bank, first=False):
        if not first:
            pltpu.make_async_copy(gbufs[1 - b], acc.at[idds[(bank - 1) % 4]],
                                  sss[1 - b]).wait()

        @pl.when(k + 1 < NCH)
        def _():
            _wait_idx(k + 1, 1 - b, (bank + 1) % 4)
            _issue_data(k + 1, 1 - b)

        pltpu.make_async_copy(q_hbm.at[ids[b]], gbufs[b], sgs[b]).wait()
        pltpu.make_async_copy(rc_hbm.at[pl.ds(ebase + k * C, C), pl.ds(0, EMB)],
                              rbufs[b], srs[b]).wait()
        g, r = gbufs[b], rbufs[b]

        @plsc.parallel_loop(0, C, 1, unroll=4)
        def cw(e):
            for j in range(EMB // LN):
                g[e, pl.ds(j * LN, LN)] = jnp.maximum(
                    g[e, pl.ds(j * LN, LN)] + r[e, pl.ds(j * LN, LN)], 0.0)

        pltpu.async_copy(g, acc.at[idds[bank]], sss[b], add=True)

        @pl.when(k + 2 < NCH)
        def _():
            _issue_idx(k + 2, b, (bank + 2) % 4)

    _step(jnp.int32(0), 0, 0, first=True)

    def body(k4, _):
        for t in range(4):
            k = 1 + 4 * k4 + t
            _step(k, (1 + t) % 2, (1 + t) % 4)
        return 0

    lax.fori_loop(0, (NCH - 1) // 4, body, 0)
    pltpu.make_async_copy(gbufs[0], acc.at[idds[0]], sss[0]).wait()
    plsc.subcore_barrier()

    def ocopy(k2, _):
        for t in range(2):
            k = 2 * k2 + t
            g = gbufs[t]

            @pl.when(k >= 2)
            def _():
                pltpu.make_async_copy(
                    g, out_hbm.at[c, pl.ds(row0 + (k - 2) * C, C)],
                    sgs[t]).wait()

            pltpu.sync_copy(acc.at[pl.ds(row0 + k * C, C)], g)
            pltpu.async_copy(g, out_hbm.at[c, pl.ds(row0 + k * C, C)],
                             sgs[t])
        return 0

    lax.fori_loop(0, ROWS_PT // (2 * C), ocopy, 0)
    for t in range(2):
        k = ROWS_PT // C - 2 + t
        pltpu.make_async_copy(gbufs[t],
                              out_hbm.at[c, pl.ds(row0 + k * C, C)],
                              sgs[t]).wait()


@functools.partial(
    pl.kernel,
    out_type=jax.ShapeDtypeStruct((E, 2 * EDGE), jnp.float32),
    mesh=_mesh,
    scratch_types=[
        pltpu.VMEM((EPW,), jnp.int32),
        pltpu.VMEM((EPW,), jnp.int32),
        pltpu.VMEM((C, EMB), jnp.float32),
        pltpu.VMEM((C, EMB), jnp.float32),
        pltpu.VMEM((C, EMB), jnp.float32),
        pltpu.VMEM((C, EMB), jnp.float32),
        pltpu.VMEM((C, EDGE), jnp.float32),
        pltpu.VMEM((C, EDGE), jnp.float32),
        pltpu.VMEM((C, 2 * EDGE), jnp.float32),
        pltpu.VMEM((C, 2 * EDGE), jnp.float32),
        pltpu.SemaphoreType.DMA,
        pltpu.SemaphoreType.DMA,
        pltpu.SemaphoreType.DMA,
        pltpu.SemaphoreType.DMA,
        pltpu.SemaphoreType.DMA,
        pltpu.SemaphoreType.DMA,
        pltpu.SemaphoreType.DMA,
        pltpu.SemaphoreType.DMA,
    ],
)
def _sc_edge_update(ts_hbm, td_hbm, rc_hbm, src_hbm, dst_hbm, ea_hbm,
                    idx_s, idx_d, sbuf0, sbuf1, dbuf0, dbuf1,
                    cbuf0, cbuf1, eabuf0, eabuf1,
                    ss0, ss1, sd0, sd1, sc0, sc1, so0, so1):
    cc = lax.axis_index("c")
    s = lax.axis_index("s")
    wid = cc * NS + s
    ebase = wid * EPW
    sbufs, dbufs = (sbuf0, sbuf1), (dbuf0, dbuf1)
    cbufs, eabufs = (cbuf0, cbuf1), (eabuf0, eabuf1)
    sss, sds, scs = (ss0, ss1), (sd0, sd1), (sc0, sc1)
    sos = (so0, so1)

    pltpu.sync_copy(src_hbm.at[pl.ds(ebase, EPW)], idx_s)
    pltpu.sync_copy(dst_hbm.at[pl.ds(ebase, EPW)], idx_d)

    def _issue(k, b):
        pltpu.async_copy(ts_hbm.at[idx_s.at[pl.ds(k * C, C)]], sbufs[b],
                         sss[b])
        pltpu.async_copy(td_hbm.at[idx_d.at[pl.ds(k * C, C)]], dbufs[b],
                         sds[b])
        pltpu.async_copy(rc_hbm.at[pl.ds(ebase + k * C, C), pl.ds(EMB, EDGE)],
                         cbufs[b], scs[b])

    def _step(k, b, tail=False):
        if not tail:
            _issue(k + 1, 1 - b)

        pltpu.make_async_copy(ts_hbm.at[idx_s.at[pl.ds(k * C, C)]], sbufs[b],
                              sss[b]).wait()
        pltpu.make_async_copy(td_hbm.at[idx_d.at[pl.ds(k * C, C)]], dbufs[b],
                              sds[b]).wait()
        pltpu.make_async_copy(rc_hbm.at[pl.ds(ebase + k * C, C), pl.ds(EMB, EDGE)],
                              cbufs[b], scs[b]).wait()
        sb, db, cb, eb = sbufs[b], dbufs[b], cbufs[b], eabufs[b]

        @pl.when(k >= 2)
        def _():
            pltpu.make_async_copy(
                eb, ea_hbm.at[pl.ds(ebase + (k - 2) * C, C)], sos[b]).wait()

        @plsc.parallel_loop(0, C, 1, unroll=4)
        def cw(e):
            eb[e, pl.ds(0, LN)] = jnp.maximum(
                sb[e, pl.ds(0, LN)] + db[e, pl.ds(0, LN)]
                + cb[e, pl.ds(0, LN)], 0.0)
            eb[e, pl.ds(LN, LN)] = (sb[e, pl.ds(LN, LN)]
                                    + db[e, pl.ds(LN, LN)])
        pltpu.async_copy(eb, ea_hbm.at[pl.ds(ebase + k * C, C)], sos[b])

    _issue(0, 0)

    def body(k2, _):
        _step(2 * k2, 0)
        _step(2 * k2 + 1, 1)
        return 0

    lax.fori_loop(0, NCH // 2, body, 0)
    _step(jnp.int32(NCH - 1), (NCH - 1) % 2, tail=True)
    pltpu.make_async_copy(
        eabufs[(NCH - 2) % 2],
        ea_hbm.at[pl.ds(ebase + (NCH - 2) * C, C)], sos[(NCH - 2) % 2]).wait()
    pltpu.make_async_copy(
        eabufs[(NCH - 1) % 2],
        ea_hbm.at[pl.ds(ebase + (NCH - 1) * C, C)], sos[(NCH - 1) % 2]).wait()


NB = 1000
EB = 16000


def _rows(bshape):
    return pl.BlockSpec(bshape, lambda i: (i,) + (0,) * (len(bshape) - 1))


def _whole(shape):
    return pl.BlockSpec(shape, lambda i: (0,) * len(shape))


def _tc_node0(x, emb_W, emb_b, Wq):
    def body(x_r, W_r, b_r, Wq_r, h_r, q_r):
        h = jnp.dot(x_r[...], W_r[...], preferred_element_type=jnp.float32) + b_r[...]
        h_r[...] = h
        q_r[...] = jnp.dot(h, Wq_r[...], preferred_element_type=jnp.float32)

    return pl.pallas_call(
        body,
        grid=(N // NB,),
        in_specs=[_rows((NB, IN)), _whole((IN, EMB)), _whole((1, EMB)),
                  _whole((EMB, EMB))],
        out_specs=[_rows((NB, EMB)), _rows((NB, EMB))],
        out_shape=[jax.ShapeDtypeStruct((N, EMB), jnp.float32)] * 2,
    )(x, emb_W, emb_b.reshape(1, -1), Wq)


def _tc_edge_pre_t(eaT, Wme, bm, Wce, bc):
    dn = (((0,), (0,)), ((), ()))

    def body(ea_r, Wm_r, bm_r, Wc_r, bc_r, rc_r):
        a = ea_r[...]
        rc_r[:, :EMB] = lax.dot_general(
            a, Wm_r[...], dn, preferred_element_type=jnp.float32) + bm_r[...]
        rc_r[:, EMB:] = lax.dot_general(
            a, Wc_r[...], dn, preferred_element_type=jnp.float32) + bc_r[...]

    return pl.pallas_call(
        body,
        grid=(E // EB,),
        in_specs=[pl.BlockSpec((EDGE, EB), lambda i: (0, i)),
                  _whole((EDGE, EMB)), _whole((1, EMB)),
                  _whole((EDGE, EDGE)), _whole((1, EDGE))],
        out_specs=_rows((EB, EMB + EDGE)),
        out_shape=jax.ShapeDtypeStruct((E, EMB + EDGE), jnp.float32),
    )(eaT, Wme, bm.reshape(1, -1), Wce, bc.reshape(1, -1))


def _tc_edge_pre(ea, Wme, bm, Wce, bc):
    W = ea.shape[1]

    def body(ea_r, Wm_r, bm_r, Wc_r, bc_r, rc_r):
        a = ea_r[:, :EDGE]
        rc_r[:, :EMB] = jnp.dot(a, Wm_r[...],
                                preferred_element_type=jnp.float32) + bm_r[...]
        rc_r[:, EMB:] = jnp.dot(a, Wc_r[...],
                                preferred_element_type=jnp.float32) + bc_r[...]

    return pl.pallas_call(
        body,
        grid=(E // EB,),
        in_specs=[_rows((EB, W)), _whole((EDGE, EMB)), _whole((1, EMB)),
                  _whole((EDGE, EDGE)), _whole((1, EDGE))],
        out_specs=_rows((EB, EMB + EDGE)),
        out_shape=jax.ShapeDtypeStruct((E, EMB + EDGE), jnp.float32),
    )(ea, Wme, bm.reshape(1, -1), Wce, bc.reshape(1, -1))


def _tc_node_update(aggr, h, We_a, We_h, be, Wq, Wts, Wtd):
    def body(a0_r, a1_r, h_r, Wa_r, Wh_r, b_r, Wq_r, Ws_r, Wd_r,
             hn_r, q_r, Ts_r, Td_r):
        a = a0_r[...] + a1_r[...]
        hn = jnp.maximum(
            jnp.dot(a, Wa_r[...], preferred_element_type=jnp.float32)
            + jnp.dot(h_r[...], Wh_r[...], preferred_element_type=jnp.float32)
            + b_r[...], 0.0)
        hn_r[...] = hn
        q_r[...] = jnp.dot(hn, Wq_r[...], preferred_element_type=jnp.float32)
        Ts_r[...] = jnp.dot(hn, Ws_r[...], preferred_element_type=jnp.float32)
        Td_r[...] = jnp.dot(hn, Wd_r[...], preferred_element_type=jnp.float32)

    return pl.pallas_call(
        body,
        grid=(N // NB,),
        in_specs=[_rows((NB, EMB)), _rows((NB, EMB)), _rows((NB, EMB)),
                  _whole((EMB, EMB)), _whole((EMB, EMB)), _whole((1, EMB)),
                  _whole((EMB, EMB)), _whole((EMB, EMB)), _whole((EMB, EMB))],
        out_specs=[_rows((NB, EMB)), _rows((NB, EMB)),
                   _rows((NB, EMB)), _rows((NB, EMB))],
        out_shape=[jax.ShapeDtypeStruct((N, EMB), jnp.float32)] * 4,
    )(aggr[0], aggr[1], h, We_a, We_h, be.reshape(1, -1), Wq, Wts, Wtd)


def _tc_node_final(aggr, h, We_a, We_h, be, node_W, node_b, Wts, Wtd):
    def body(a0_r, a1_r, h_r, Wa_r, Wh_r, b_r, Wn_r, bn_r, Ws_r, Wd_r,
             np_r, Ts_r, Td_r):
        a = a0_r[...] + a1_r[...]
        h2 = jnp.maximum(
            jnp.dot(a, Wa_r[...], preferred_element_type=jnp.float32)
            + jnp.dot(h_r[...], Wh_r[...], preferred_element_type=jnp.float32)
            + b_r[...], 0.0)
        np_r[...] = jnp.dot(h2, Wn_r[...],
                            preferred_element_type=jnp.float32) + bn_r[...]
        Ts_r[...] = jnp.dot(h2, Ws_r[...], preferred_element_type=jnp.float32)
        Td_r[...] = jnp.dot(h2, Wd_r[...], preferred_element_type=jnp.float32)

    return pl.pallas_call(
        body,
        grid=(N // NB,),
        in_specs=[_rows((NB, EMB)), _rows((NB, EMB)), _rows((NB, EMB)),
                  _whole((EMB, EMB)), _whole((EMB, EMB)), _whole((1, EMB)),
                  _whole((EMB, OUT)), _whole((1, OUT)),
                  _whole((EMB, EMB)), _whole((EMB, EMB))],
        out_specs=[_rows((NB, OUT)), _rows((NB, EMB)), _rows((NB, EMB))],
        out_shape=[jax.ShapeDtypeStruct((N, OUT), jnp.float32),
                   jax.ShapeDtypeStruct((N, EMB), jnp.float32),
                   jax.ShapeDtypeStruct((N, EMB), jnp.float32)],
    )(aggr[0], aggr[1], h, We_a, We_h, be.reshape(1, -1),
      node_W, node_b.reshape(1, -1), Wts, Wtd)


def _tc_edge_final(ea2e, ep_We, ep_b):
    w17 = jnp.concatenate([ep_We[:, 0], jnp.ones((1,), jnp.float32)])
    w17 = w17.reshape(1, EDGE + 1)
    dn = (((1,), (1,)), ((), ()))

    def body(ea_r, Wp_r, bp_r, ep_r):
        ep_r[...] = lax.dot_general(
            Wp_r[...], ea_r[:, :EDGE + 1], dn,
            preferred_element_type=jnp.float32) + bp_r[...]

    return pl.pallas_call(
        body,
        grid=(E // EB,),
        in_specs=[_rows((EB, 2 * EDGE)), _whole((1, EDGE + 1)),
                  _whole((1, 1))],
        out_specs=pl.BlockSpec((1, EB), lambda i: (0, i)),
        out_shape=jax.ShapeDtypeStruct((1, E), jnp.float32),
    )(ea2e, w17, ep_b.reshape(1, 1))


def kernel(x, edge_attr, edge_index, emb_W, emb_b, Wm, bm, We, be,
           Weu, beu, node_W, node_b, ep_W, ep_b):
    src = edge_index[0]
    dst = edge_index[1]

    Wm_h = [Wm[i, :EMB] for i in range(2)]
    Wm_e = [Wm[i, EMB:] for i in range(2)]
    We_a = [We[i, :EMB] for i in range(2)]
    We_h = [We[i, EMB:] for i in range(2)]
    Weu_s = [Weu[i, :EMB] for i in range(2)]
    Weu_d = [Weu[i, EMB:2 * EMB] for i in range(2)]
    Weu_e = [Weu[i, 2 * EMB:] for i in range(2)]
    zcol = jnp.zeros((EMB, 1), jnp.float32)
    zpad = jnp.zeros((EMB, EMB - EDGE - 1), jnp.float32)
    Wts1 = jnp.concatenate([Weu_s[0], zcol, zpad], axis=1)
    Wtd1 = jnp.concatenate([Weu_d[0], zcol, zpad], axis=1)
    Wts2 = jnp.concatenate([Weu_s[1], ep_W[:EMB], zpad], axis=1)
    Wtd2 = jnp.concatenate([Weu_d[1], ep_W[EMB:2 * EMB], zpad], axis=1)
    ep_We = ep_W[2 * EMB:]

    h0, q0 = _tc_node0(x, emb_W, emb_b, Wm_h[0])
    rc0 = _tc_edge_pre_t(edge_attr.T, Wm_e[0], bm[0], Weu_e[0], beu[0])
    aggr0 = _sc_msg_aggr(q0, rc0, src, dst)
    h1, q1, T1s, T1d = _tc_node_update(aggr0, h0, We_a[0], We_h[0], be[0],
                                       Wm_h[1], Wts1, Wtd1)
    ea1e = _sc_edge_update(T1s, T1d, rc0, src, dst)

    rc1 = _tc_edge_pre(ea1e, Wm_e[1], bm[1], Weu_e[1], beu[1])
    aggr1 = _sc_msg_aggr(q1, rc1, src, dst)
    node_pred, T2s, T2d = _tc_node_final(aggr1, h1, We_a[1], We_h[1], be[1],
                                         node_W, node_b, Wts2, Wtd2)
    ea2e = _sc_edge_update(T2s, T2d, rc1, src, dst)
    edge_pred = _tc_edge_final(ea2e, ep_We, ep_b).reshape(E, 1)

    return (node_pred, edge_pred)
